# baseline (speedup 1.0000x reference)
import functools

import jax
import jax.numpy as jnp
from jax import lax
from jax.experimental import pallas as pl
from jax.experimental.pallas import tpu as pltpu
from jax.experimental.pallas import tpu_sc as plsc

N, D, E, DFF = 2048, 768, 8, 3072
TB = 128
R = N // TB
W = R + E - 1
TBA = 256
K = 16
C = N // K
F32 = jnp.float32


def _lnr_body(x_ref, g_ref, b_ref, rw_ref, rb_ref,
              gate_ref, idx_ref, rank_ref, cnt_ref, acc_ref):
    i = pl.program_id(0)
    x = x_ref[...]
    mu = jnp.mean(x, axis=1, keepdims=True)
    var = jnp.mean((x - mu) ** 2, axis=1, keepdims=True)
    xn = (x - mu) / jnp.sqrt(var + 1e-5) * g_ref[...] + b_ref[...]
    logits = jnp.dot(xn, rw_ref[...], preferred_element_type=F32) + rb_ref[...]
    m = jnp.max(logits, axis=1, keepdims=True)
    gate_ref[...] = 1.0 / jnp.sum(jnp.exp(logits - m), axis=1, keepdims=True)
    e_iota = lax.broadcasted_iota(jnp.int32, (TBA, E), 1)
    idxv = jnp.min(jnp.where(logits >= m, e_iota, E), axis=1, keepdims=True)
    idx_ref[...] = idxv.astype(jnp.int32)
    onehot = (idxv == e_iota).astype(F32)
    r1 = lax.broadcasted_iota(jnp.int32, (TBA, TBA), 0)
    c1 = lax.broadcasted_iota(jnp.int32, (TBA, TBA), 1)
    tri = (c1 <= r1).astype(F32)
    cum = jnp.dot(tri, onehot, preferred_element_type=F32)

    @pl.when(i == 0)
    def _():
        acc_ref[...] = jnp.zeros((1, E), F32)

    base = acc_ref[...]
    rank = (jnp.sum(onehot * (cum + base), axis=1, keepdims=True) - 1.0)
    rank_ref[...] = rank.astype(jnp.int32)
    newbase = base + cum[TBA - 1:TBA, :]
    acc_ref[...] = newbase
    cnt_ref[...] = newbase.astype(jnp.int32)


def _ln_router(x2, g, b, rw, rb):
    return pl.pallas_call(
        _lnr_body,
        grid=(N // TBA,),
        in_specs=[
            pl.BlockSpec((TBA, D), lambda i: (i, 0)),
            pl.BlockSpec((1, D), lambda i: (0, 0)),
            pl.BlockSpec((1, D), lambda i: (0, 0)),
            pl.BlockSpec((D, E), lambda i: (0, 0)),
            pl.BlockSpec((1, E), lambda i: (0, 0)),
        ],
        out_specs=[
            pl.BlockSpec((TBA, 1), lambda i: (i, 0)),
            pl.BlockSpec((TBA, 1), lambda i: (i, 0)),
            pl.BlockSpec((TBA, 1), lambda i: (i, 0)),
            pl.BlockSpec((1, E), lambda i: (0, 0)),
        ],
        out_shape=[
            jax.ShapeDtypeStruct((N, 1), F32),
            jax.ShapeDtypeStruct((N, 1), jnp.int32),
            jax.ShapeDtypeStruct((N, 1), jnp.int32),
            jax.ShapeDtypeStruct((1, E), jnp.int32),
        ],
        scratch_shapes=[pltpu.VMEM((1, E), F32)],
    )(x2, g, b, rw, rb)


def _route_meta(idx_flat, rank_flat, counts):
    off = jnp.concatenate([jnp.zeros((1,), jnp.int32),
                           jnp.cumsum(counts[0])]).astype(jnp.int32)
    s = off[idx_flat] + rank_flat
    starts = jnp.sort(jnp.concatenate(
        [off[1:E], jnp.arange(R, dtype=jnp.int32) * TB]))
    r_ids = jnp.clip(starts // TB, 0, R - 1).astype(jnp.int32)
    e_ids = jnp.clip(
        jnp.sum(starts[:, None] >= off[None, :], axis=1, dtype=jnp.int32) - 1,
        0, E - 1)
    return s.astype(jnp.int32), r_ids, e_ids, off


_NC, _NS = 2, 16
_NW = _NC * _NS
_BPW = N // _NW


def _sc_gather(xn, perm):
    mesh = plsc.VectorSubcoreMesh(core_axis_name="c", subcore_axis_name="s")

    @functools.partial(
        pl.kernel, mesh=mesh,
        out_type=jax.ShapeDtypeStruct((N, D), F32),
        scratch_types=[
            pltpu.VMEM((_BPW,), jnp.int32),
            pltpu.VMEM((_BPW, D), F32),
            pltpu.SemaphoreType.DMA,
        ],
    )
    def k(xn_hbm, perm_hbm, out_hbm, idx_v, rows_v, sem):
        wid = lax.axis_index("s") * _NC + lax.axis_index("c")
        base = wid * _BPW
        pltpu.sync_copy(perm_hbm.at[pl.ds(base, _BPW)], idx_v)
        pltpu.async_copy(xn_hbm.at[idx_v], rows_v, sem).wait()
        pltpu.sync_copy(rows_v, out_hbm.at[pl.ds(base, _BPW)])

    return k(xn, perm)


def _sc_scatter(ys, perm):
    mesh = plsc.VectorSubcoreMesh(core_axis_name="c", subcore_axis_name="s")

    @functools.partial(
        pl.kernel, mesh=mesh,
        out_type=jax.ShapeDtypeStruct((N, D), F32),
        scratch_types=[
            pltpu.VMEM((_BPW,), jnp.int32),
            pltpu.VMEM((_BPW, D), F32),
            pltpu.SemaphoreType.DMA,
        ],
    )
    def k(ys_hbm, perm_hbm, out_hbm, idx_v, rows_v, sem):
        wid = lax.axis_index("s") * _NC + lax.axis_index("c")
        base = wid * _BPW
        pltpu.sync_copy(perm_hbm.at[pl.ds(base, _BPW)], idx_v)
        pltpu.sync_copy(ys_hbm.at[pl.ds(base, _BPW)], rows_v)
        pltpu.async_copy(rows_v, out_hbm.at[idx_v], sem).wait()

    return k(ys, perm)


def _ffn_body(r_ref, e_ref, off_ref,
              xs_ref, g_ref, b_ref, w1_ref, b1_ref, w2_ref, b2_ref, out_ref):
    j = pl.program_id(0)
    r = r_ref[j]
    e = e_ref[j]
    x = xs_ref[...]
    mu = jnp.mean(x, axis=1, keepdims=True)
    var = jnp.mean((x - mu) ** 2, axis=1, keepdims=True)
    xn = (x - mu) / jnp.sqrt(var + 1e-5) * g_ref[...] + b_ref[...]
    h = jnp.dot(xn, w1_ref[0], preferred_element_type=F32) + b1_ref[0]
    h = jax.nn.gelu(h)
    y = jnp.dot(h, w2_ref[0], preferred_element_type=F32) + b2_ref[0]
    jp = jnp.maximum(j - 1, 0)
    dup = (j > 0) & (r == r_ref[jp]) & (e == e_ref[jp])
    lo = jnp.maximum(off_ref[e], r * TB)
    hi = jnp.where(dup, lo, jnp.minimum(off_ref[e + 1], (r + 1) * TB))
    rows = r * TB + lax.broadcasted_iota(jnp.int32, (TB, 1), 0)
    mask = (rows >= lo) & (rows < hi)
    contrib = jnp.where(mask, y, 0.0)
    first = jnp.logical_or(j == 0, r != r_ref[jp])

    @pl.when(first)
    def _():
        out_ref[...] = contrib

    @pl.when(jnp.logical_not(first))
    def _():
        out_ref[...] += contrib


def _ffn_grouped(r_ids, e_ids, off, xs, g, b, w1, b1, w2, b2):
    grid_spec = pltpu.PrefetchScalarGridSpec(
        num_scalar_prefetch=3,
        grid=(W,),
        in_specs=[
            pl.BlockSpec((TB, D), lambda j, r, e, off: (r[j], 0)),
            pl.BlockSpec((1, D), lambda j, r, e, off: (0, 0)),
            pl.BlockSpec((1, D), lambda j, r, e, off: (0, 0)),
            pl.BlockSpec((1, D, DFF), lambda j, r, e, off: (e[j], 0, 0)),
            pl.BlockSpec((1, 1, DFF), lambda j, r, e, off: (e[j], 0, 0)),
            pl.BlockSpec((1, DFF, D), lambda j, r, e, off: (e[j], 0, 0)),
            pl.BlockSpec((1, 1, D), lambda j, r, e, off: (e[j], 0, 0)),
        ],
        out_specs=pl.BlockSpec((TB, D), lambda j, r, e, off: (r[j], 0)),
    )
    return pl.pallas_call(
        _ffn_body,
        grid_spec=grid_spec,
        out_shape=jax.ShapeDtypeStruct((N, D), F32),
    )(r_ids, e_ids, off, xs, g, b, w1, b1.reshape(E, 1, DFF), w2,
      b2.reshape(E, 1, D))


_ID8 = (1.0, 0.0, 0.0, 0.0, 0.0, 0.0, 1.0, 0.0)


def _mm2x2(a, b):
    (a00r, a00i, a01r, a01i, a10r, a10i, a11r, a11i) = a
    (b00r, b00i, b01r, b01i, b10r, b10i, b11r, b11i) = b
    c00r = a00r * b00r - a00i * b00i + a01r * b10r - a01i * b10i
    c00i = a00r * b00i + a00i * b00r + a01r * b10i + a01i * b10r
    c01r = a00r * b01r - a00i * b01i + a01r * b11r - a01i * b11i
    c01i = a00r * b01i + a00i * b01r + a01r * b11i + a01i * b11r
    c10r = a10r * b00r - a10i * b00i + a11r * b10r - a11i * b10i
    c10i = a10r * b00i + a10i * b00r + a11r * b10i + a11i * b10r
    c11r = a10r * b01r - a10i * b01i + a11r * b11r - a11i * b11i
    c11i = a10r * b01i + a10i * b01r + a11r * b11i + a11i * b11r
    return (c00r, c00i, c01r, c01i, c10r, c10i, c11r, c11i)


def _mnorm(m):
    mx = m[0] * 0.0
    for t in m:
        mx = jnp.maximum(mx, jnp.abs(t))
    s = 1.0 / jnp.maximum(mx, 1e-30)
    return tuple(t * s for t in m)


def _shift(m, s, right):
    stacked = jnp.concatenate(m, axis=0)
    ii = lax.broadcasted_iota(jnp.int32, (C, C), 0)
    jj = lax.broadcasted_iota(jnp.int32, (C, C), 1)
    d = s if right else -s
    sel = jnp.where(jj - ii == d, 1.0, 0.0).astype(F32)
    sh = jnp.dot(stacked, sel, preferred_element_type=F32)
    lanes = lax.broadcasted_iota(jnp.int32, (1, C), 1)
    cond = (lanes < s) if right else (lanes >= C - s)
    return tuple(jnp.where(cond, idv, sh[t:t + 1])
                 for t, idv in enumerate(_ID8))


def _chain(ar, forward):
    one = jnp.ones((1, C), F32)
    zero = jnp.zeros((1, C), F32)
    order = range(K) if forward else range(K - 1, -1, -1)
    out = [None] * K
    p = None
    for k in order:
        arr = ar[k:k + 1, :]
        if p is None:
            p = (arr, -one, -one, zero, one, zero, zero, zero)
        else:
            (p00r, p00i, p01r, p01i, p10r, p10i, p11r, p11i) = p
            n00r = arr * p00r + p00i - p10r
            n00i = arr * p00i - p00r - p10i
            n01r = arr * p01r + p01i - p11r
            n01i = arr * p01i - p01r - p11i
            p = (n00r, n00i, n01r, n01i, p00r, p00i, p01r, p01i)
        out[k] = p
    return out


def _prefix(m0, forward):
    x = _mnorm(m0)
    s = 1
    while s < C:
        xs = _shift(x, s, right=forward)
        x = _mnorm(_mm2x2(x, xs))
        s *= 2
    xs = _shift(x, 1, right=forward)
    return xs[0], xs[1], xs[4], xs[5]


def _bk_body(ys_ref, gate_ref, pw_ref, pb_ref, ow_ref, ob_ref, sc_ref, out_ref):
    ys = ys_ref[...]
    moe = ys * gate_ref[...]
    v = jnp.dot(moe, pw_ref[...], preferred_element_type=F32) + pb_ref[0, 0]
    hd = jnp.clip(v, -3.0, 3.0) - 2.0
    i2 = lax.broadcasted_iota(jnp.int32, (N, C), 0)
    c2 = lax.broadcasted_iota(jnp.int32, (N, C), 1)
    sel_c = jnp.where((i2 // K) == c2, 1.0, 0.0).astype(F32)
    k16 = lax.broadcasted_iota(jnp.int32, (K, N), 0)
    i16 = lax.broadcasted_iota(jnp.int32, (K, N), 1)
    w1sel = jnp.where((i16 % K) == k16, 1.0, 0.0).astype(F32)
    ar = jnp.dot(w1sel, sel_c * hd, preferred_element_type=F32)

    P = _chain(ar, forward=True)
    Q = _chain(ar, forward=False)
    unr, uni, udr, udi = _prefix(P[K - 1], forward=True)
    wnr, wni, wdr, wdi = _prefix(Q[0], forward=False)

    g_rows_r, g_rows_i = [], []
    for k in range(K):
        (p00r, p00i, p01r, p01i, p10r, p10i, p11r, p11i) = P[k]
        nLr = p00r * unr - p00i * uni + p01r * udr - p01i * udi
        nLi = p00r * uni + p00i * unr + p01r * udi + p01i * udr
        dLr = p10r * unr - p10i * uni + p11r * udr - p11i * udi
        dLi = p10r * uni + p10i * unr + p11r * udi + p11i * udr
        dd = jnp.maximum(dLr * dLr + dLi * dLi, 1e-30)
        Lr = (nLr * dLr + nLi * dLi) / dd
        Li = (nLi * dLr - nLr * dLi) / dd
        (q00r, q00i, q01r, q01i, q10r, q10i, q11r, q11i) = Q[k]
        nRr = q00r * wnr - q00i * wni + q01r * wdr - q01i * wdi
        nRi = q00r * wni + q00i * wnr + q01r * wdi + q01i * wdr
        dRr = q10r * wnr - q10i * wni + q11r * wdr - q11i * wdi
        dRi = q10r * wni + q10i * wnr + q11r * wdi + q11i * wdr
        ddr = jnp.maximum(dRr * dRr + dRi * dRi, 1e-30)
        Rr = (nRr * dRr + nRi * dRi) / ddr
        Ri = (nRi * dRr - nRr * dRi) / ddr
        sr = Lr + Rr - ar[k:k + 1, :]
        si = Li + Ri + 1.0
        den = jnp.maximum(sr * sr + si * si, 1e-30)
        g_rows_r.append(jnp.clip(sr / den, -10.0, 10.0))
        g_rows_i.append(jnp.clip(-si / den, -10.0, 10.0))
    re_g = jnp.concatenate(g_rows_r, axis=0)
    im_g = jnp.concatenate(g_rows_i, axis=0)

    w1t = jnp.where((lax.broadcasted_iota(jnp.int32, (N, K), 0) % K)
                    == lax.broadcasted_iota(jnp.int32, (N, K), 1),
                    1.0, 0.0).astype(F32)
    col_r = jnp.sum(jnp.dot(w1t, re_g, preferred_element_type=F32) * sel_c,
                    axis=1, keepdims=True)
    col_i = jnp.sum(jnp.dot(w1t, im_g, preferred_element_type=F32) * sel_c,
                    axis=1, keepdims=True)
    spec = col_r * ow_ref[0:1, :] + col_i * ow_ref[1:2, :] + ob_ref[...]
    out_ref[...] = moe + sc_ref[0, 0] * spec


def _bk_final(yu, gate, pw, pb, ow, ob, bscale):
    return pl.pallas_call(
        _bk_body,
        grid=(1,),
        in_specs=[
            pl.BlockSpec((N, D), lambda i: (0, 0)),
            pl.BlockSpec((N, 1), lambda i: (0, 0)),
            pl.BlockSpec((D, 1), lambda i: (0, 0)),
            pl.BlockSpec((1, 1), lambda i: (0, 0)),
            pl.BlockSpec((2, D), lambda i: (0, 0)),
            pl.BlockSpec((1, D), lambda i: (0, 0)),
            pl.BlockSpec((1, 1), lambda i: (0, 0)),
        ],
        out_specs=pl.BlockSpec((N, D), lambda i: (0, 0)),
        out_shape=jax.ShapeDtypeStruct((N, D), F32),
    )(yu, gate, pw, pb, ow, ob, bscale)


def kernel(x, ln_gamma, ln_beta, router_w, router_b, w1, b1, w2, b2,
           pproj_w, pproj_b, oproj_w, oproj_b, bk_scale):
    x2 = x.reshape(N, D)
    g2, b2r = ln_gamma.reshape(1, D), ln_beta.reshape(1, D)
    gate, idx, rank, counts = _ln_router(x2, g2, b2r, router_w,
                                         router_b.reshape(1, E))
    s, r_ids, e_ids, off = _route_meta(idx.reshape(N), rank.reshape(N), counts)
    xs = _sc_scatter(x2, s)
    ys = _ffn_grouped(r_ids, e_ids, off, xs, g2, b2r, w1, b1, w2, b2)
    yu = _sc_gather(ys, s)
    out = _bk_final(yu, gate, pproj_w, pproj_b.reshape(1, 1), oproj_w,
                    oproj_b.reshape(1, D), jnp.asarray(bk_scale).reshape(1, 1))
    return out.reshape(1, N, D)


# speedup vs baseline: 1.0574x; 1.0574x over previous
import functools

import jax
import jax.numpy as jnp
from jax import lax
from jax.experimental import pallas as pl
from jax.experimental.pallas import tpu as pltpu
from jax.experimental.pallas import tpu_sc as plsc

N, D, E, DFF = 2048, 768, 8, 3072
TB = 128
R = N // TB
W = R + E - 1
TBA = 256
K = 16
C = N // K
F32 = jnp.float32


_NB = N // TBA
_VS = 32


def _lnr_body(x_ref, g_ref, b_ref, rw_ref, rb_ref,
              gate_ref, s_ref, rid_ref, eid_ref, off_ref,
              acc_ref, idx_scr, rank_scr):
    i = pl.program_id(0)
    x = x_ref[...]
    mu = jnp.mean(x, axis=1, keepdims=True)
    var = jnp.mean((x - mu) ** 2, axis=1, keepdims=True)
    xn = (x - mu) / jnp.sqrt(var + 1e-5) * g_ref[...] + b_ref[...]
    logits = jnp.dot(xn, rw_ref[...], preferred_element_type=F32) + rb_ref[...]
    m = jnp.max(logits, axis=1, keepdims=True)
    gate_ref[...] = 1.0 / jnp.sum(jnp.exp(logits - m), axis=1, keepdims=True)
    e_iota = lax.broadcasted_iota(jnp.int32, (TBA, E), 1)
    idxv = jnp.min(jnp.where(logits >= m, e_iota, E), axis=1, keepdims=True)
    onehot = (idxv == e_iota).astype(F32)
    r1 = lax.broadcasted_iota(jnp.int32, (TBA, TBA), 0)
    c1 = lax.broadcasted_iota(jnp.int32, (TBA, TBA), 1)
    tri = (c1 <= r1).astype(F32)
    cum = jnp.dot(tri, onehot, preferred_element_type=F32)

    @pl.when(i == 0)
    def _():
        acc_ref[...] = jnp.zeros((1, E), F32)

    base = acc_ref[...]
    rank = jnp.sum(onehot * (cum + base), axis=1, keepdims=True) - 1.0
    idx_scr[pl.ds(i * TBA, TBA), :] = idxv.astype(jnp.int32)
    rank_scr[pl.ds(i * TBA, TBA), :] = rank
    acc_ref[...] = base + cum[TBA - 1:TBA, :]

    @pl.when(i == _NB - 1)
    def _():
        counts = acc_ref[...]
        ku = lax.broadcasted_iota(jnp.int32, (E, E), 0)
        eu = lax.broadcasted_iota(jnp.int32, (E, E), 1)
        upper = (ku < eu).astype(F32)
        off_ex = jnp.dot(counts, upper, preferred_element_type=F32)
        idx_all = idx_scr[...]
        oh_all = (idx_all == lax.broadcasted_iota(jnp.int32, (N, E), 1))
        s = rank_scr[...] + jnp.sum(oh_all.astype(F32) * off_ex, axis=1,
                                    keepdims=True)
        s_ref[...] = s.astype(jnp.int32)
        blocks = (lax.broadcasted_iota(jnp.int32, (1, R), 1) * TB).astype(F32)
        huge = jnp.full((1, _VS - (E - 1) - R), 1e9, F32)
        vals = jnp.concatenate([off_ex[:, 1:E], blocks, huge], axis=1)
        ident = (lax.broadcasted_iota(jnp.int32, (_VS, _VS), 0)
                 == lax.broadcasted_iota(jnp.int32, (_VS, _VS), 1)).astype(F32)
        vals_col = lax.dot_general(ident, vals, (((1,), (1,)), ((), ())),
                                   preferred_element_type=F32)
        ii = lax.broadcasted_iota(jnp.int32, (_VS, _VS), 0)
        jj = lax.broadcasted_iota(jnp.int32, (_VS, _VS), 1)
        less = (vals < vals_col) | ((vals == vals_col) & (jj < ii))
        pos = jnp.sum(less.astype(F32), axis=1, keepdims=True)
        pmat = (pos == jj.astype(F32)).astype(F32)
        sortv = lax.dot_general(vals, pmat, (((1,), (0,)), ((), ())),
                                preferred_element_type=F32)
        rid = jnp.clip(jnp.floor(sortv * (1.0 / TB)), 0.0, R - 1.0)
        eacc = jnp.zeros((1, _VS), F32)
        for mth in range(E):
            eacc = eacc + (sortv >= off_ex[:, mth:mth + 1]).astype(F32)
        eacc = eacc + (sortv >= float(N)).astype(F32)
        rid_ref[...] = rid.astype(jnp.int32)
        eid_ref[...] = jnp.clip(eacc - 1.0, 0.0, E - 1.0).astype(jnp.int32)
        offp = jnp.concatenate(
            [off_ex, jnp.full((1, 1), float(N), F32),
             jnp.zeros((1, 16 - E - 1), F32)], axis=1)
        off_ref[...] = offp.astype(jnp.int32)


def _ln_router(x2, g, b, rw, rb):
    return pl.pallas_call(
        _lnr_body,
        grid=(_NB,),
        in_specs=[
            pl.BlockSpec((TBA, D), lambda i: (i, 0)),
            pl.BlockSpec((1, D), lambda i: (0, 0)),
            pl.BlockSpec((1, D), lambda i: (0, 0)),
            pl.BlockSpec((D, E), lambda i: (0, 0)),
            pl.BlockSpec((1, E), lambda i: (0, 0)),
        ],
        out_specs=[
            pl.BlockSpec((TBA, 1), lambda i: (i, 0)),
            pl.BlockSpec((N, 1), lambda i: (0, 0)),
            pl.BlockSpec((1, _VS), lambda i: (0, 0)),
            pl.BlockSpec((1, _VS), lambda i: (0, 0)),
            pl.BlockSpec((1, 16), lambda i: (0, 0)),
        ],
        out_shape=[
            jax.ShapeDtypeStruct((N, 1), F32),
            jax.ShapeDtypeStruct((N, 1), jnp.int32),
            jax.ShapeDtypeStruct((1, _VS), jnp.int32),
            jax.ShapeDtypeStruct((1, _VS), jnp.int32),
            jax.ShapeDtypeStruct((1, 16), jnp.int32),
        ],
        scratch_shapes=[pltpu.VMEM((1, E), F32),
                        pltpu.VMEM((N, 1), jnp.int32),
                        pltpu.VMEM((N, 1), F32)],
    )(x2, g, b, rw, rb)


_NC, _NS = 2, 16
_NW = _NC * _NS
_BPW = N // _NW


def _sc_gather(xn, perm):
    mesh = plsc.VectorSubcoreMesh(core_axis_name="c", subcore_axis_name="s")

    @functools.partial(
        pl.kernel, mesh=mesh,
        out_type=jax.ShapeDtypeStruct((N, D), F32),
        scratch_types=[
            pltpu.VMEM((_BPW,), jnp.int32),
            pltpu.VMEM((_BPW, D), F32),
            pltpu.SemaphoreType.DMA,
        ],
    )
    def k(xn_hbm, perm_hbm, out_hbm, idx_v, rows_v, sem):
        wid = lax.axis_index("s") * _NC + lax.axis_index("c")
        base = wid * _BPW
        pltpu.sync_copy(perm_hbm.at[pl.ds(base, _BPW)], idx_v)
        pltpu.async_copy(xn_hbm.at[idx_v], rows_v, sem).wait()
        pltpu.sync_copy(rows_v, out_hbm.at[pl.ds(base, _BPW)])

    return k(xn, perm)


def _sc_scatter(ys, perm):
    mesh = plsc.VectorSubcoreMesh(core_axis_name="c", subcore_axis_name="s")

    @functools.partial(
        pl.kernel, mesh=mesh,
        out_type=jax.ShapeDtypeStruct((N, D), F32),
        scratch_types=[
            pltpu.VMEM((_BPW,), jnp.int32),
            pltpu.VMEM((_BPW, D), F32),
            pltpu.SemaphoreType.DMA,
        ],
    )
    def k(ys_hbm, perm_hbm, out_hbm, idx_v, rows_v, sem):
        wid = lax.axis_index("s") * _NC + lax.axis_index("c")
        base = wid * _BPW
        pltpu.sync_copy(perm_hbm.at[pl.ds(base, _BPW)], idx_v)
        pltpu.sync_copy(ys_hbm.at[pl.ds(base, _BPW)], rows_v)
        pltpu.async_copy(rows_v, out_hbm.at[idx_v], sem).wait()

    return k(ys, perm)


def _ffn_body(r_ref, e_ref, off_ref,
              xs_ref, g_ref, b_ref, w1_ref, b1_ref, w2_ref, b2_ref, out_ref):
    j = pl.program_id(0)
    r = r_ref[0, j]
    e = e_ref[0, j]
    x = xs_ref[...]
    mu = jnp.mean(x, axis=1, keepdims=True)
    var = jnp.mean((x - mu) ** 2, axis=1, keepdims=True)
    xn = (x - mu) / jnp.sqrt(var + 1e-5) * g_ref[...] + b_ref[...]
    h = jnp.dot(xn, w1_ref[0], preferred_element_type=F32) + b1_ref[0]
    h = jax.nn.gelu(h)
    y = jnp.dot(h, w2_ref[0], preferred_element_type=F32) + b2_ref[0]
    jp = jnp.maximum(j - 1, 0)
    dup = (j > 0) & (r == r_ref[0, jp]) & (e == e_ref[0, jp])
    lo = jnp.maximum(off_ref[0, e], r * TB)
    hi = jnp.where(dup, lo, jnp.minimum(off_ref[0, e + 1], (r + 1) * TB))
    rows = r * TB + lax.broadcasted_iota(jnp.int32, (TB, 1), 0)
    mask = (rows >= lo) & (rows < hi)
    contrib = jnp.where(mask, y, 0.0)
    first = jnp.logical_or(j == 0, r != r_ref[0, jp])

    @pl.when(first)
    def _():
        out_ref[...] = contrib

    @pl.when(jnp.logical_not(first))
    def _():
        out_ref[...] += contrib


def _ffn_grouped(r_ids, e_ids, off, xs, g, b, w1, b1, w2, b2):
    grid_spec = pltpu.PrefetchScalarGridSpec(
        num_scalar_prefetch=3,
        grid=(W,),
        in_specs=[
            pl.BlockSpec((TB, D), lambda j, r, e, off: (r[0, j], 0)),
            pl.BlockSpec((1, D), lambda j, r, e, off: (0, 0)),
            pl.BlockSpec((1, D), lambda j, r, e, off: (0, 0)),
            pl.BlockSpec((1, D, DFF), lambda j, r, e, off: (e[0, j], 0, 0)),
            pl.BlockSpec((1, 1, DFF), lambda j, r, e, off: (e[0, j], 0, 0)),
            pl.BlockSpec((1, DFF, D), lambda j, r, e, off: (e[0, j], 0, 0)),
            pl.BlockSpec((1, 1, D), lambda j, r, e, off: (e[0, j], 0, 0)),
        ],
        out_specs=pl.BlockSpec((TB, D), lambda j, r, e, off: (r[0, j], 0)),
    )
    return pl.pallas_call(
        _ffn_body,
        grid_spec=grid_spec,
        out_shape=jax.ShapeDtypeStruct((N, D), F32),
    )(r_ids, e_ids, off, xs, g, b, w1, b1.reshape(E, 1, DFF), w2,
      b2.reshape(E, 1, D))


_ID8 = (1.0, 0.0, 0.0, 0.0, 0.0, 0.0, 1.0, 0.0)


def _mm2x2(a, b):
    (a00r, a00i, a01r, a01i, a10r, a10i, a11r, a11i) = a
    (b00r, b00i, b01r, b01i, b10r, b10i, b11r, b11i) = b
    c00r = a00r * b00r - a00i * b00i + a01r * b10r - a01i * b10i
    c00i = a00r * b00i + a00i * b00r + a01r * b10i + a01i * b10r
    c01r = a00r * b01r - a00i * b01i + a01r * b11r - a01i * b11i
    c01i = a00r * b01i + a00i * b01r + a01r * b11i + a01i * b11r
    c10r = a10r * b00r - a10i * b00i + a11r * b10r - a11i * b10i
    c10i = a10r * b00i + a10i * b00r + a11r * b10i + a11i * b10r
    c11r = a10r * b01r - a10i * b01i + a11r * b11r - a11i * b11i
    c11i = a10r * b01i + a10i * b01r + a11r * b11i + a11i * b11r
    return (c00r, c00i, c01r, c01i, c10r, c10i, c11r, c11i)


def _mnorm(m):
    mx = m[0] * 0.0
    for t in m:
        mx = jnp.maximum(mx, jnp.abs(t))
    s = 1.0 / jnp.maximum(mx, 1e-30)
    return tuple(t * s for t in m)


def _shift(m, s, right):
    stacked = jnp.concatenate(m, axis=0)
    ii = lax.broadcasted_iota(jnp.int32, (C, C), 0)
    jj = lax.broadcasted_iota(jnp.int32, (C, C), 1)
    d = s if right else -s
    sel = jnp.where(jj - ii == d, 1.0, 0.0).astype(F32)
    sh = jnp.dot(stacked, sel, preferred_element_type=F32)
    lanes = lax.broadcasted_iota(jnp.int32, (1, C), 1)
    cond = (lanes < s) if right else (lanes >= C - s)
    return tuple(jnp.where(cond, idv, sh[t:t + 1])
                 for t, idv in enumerate(_ID8))


def _chain(ar, forward):
    one = jnp.ones((1, C), F32)
    zero = jnp.zeros((1, C), F32)
    order = range(K) if forward else range(K - 1, -1, -1)
    out = [None] * K
    p = None
    for k in order:
        arr = ar[k:k + 1, :]
        if p is None:
            p = (arr, -one, -one, zero, one, zero, zero, zero)
        else:
            (p00r, p00i, p01r, p01i, p10r, p10i, p11r, p11i) = p
            n00r = arr * p00r + p00i - p10r
            n00i = arr * p00i - p00r - p10i
            n01r = arr * p01r + p01i - p11r
            n01i = arr * p01i - p01r - p11i
            p = (n00r, n00i, n01r, n01i, p00r, p00i, p01r, p01i)
        out[k] = p
    return out


def _prefix(m0, forward):
    x = _mnorm(m0)
    s = 1
    while s < C:
        xs = _shift(x, s, right=forward)
        x = _mnorm(_mm2x2(x, xs))
        s *= 2
    xs = _shift(x, 1, right=forward)
    return xs[0], xs[1], xs[4], xs[5]


def _bk_body(ys_ref, gate_ref, pw_ref, pb_ref, ow_ref, ob_ref, sc_ref, out_ref):
    ys = ys_ref[...]
    moe = ys * gate_ref[...]
    v = jnp.dot(moe, pw_ref[...], preferred_element_type=F32) + pb_ref[0, 0]
    hd = jnp.clip(v, -3.0, 3.0) - 2.0
    i2 = lax.broadcasted_iota(jnp.int32, (N, C), 0)
    c2 = lax.broadcasted_iota(jnp.int32, (N, C), 1)
    sel_c = jnp.where((i2 // K) == c2, 1.0, 0.0).astype(F32)
    k16 = lax.broadcasted_iota(jnp.int32, (K, N), 0)
    i16 = lax.broadcasted_iota(jnp.int32, (K, N), 1)
    w1sel = jnp.where((i16 % K) == k16, 1.0, 0.0).astype(F32)
    ar = jnp.dot(w1sel, sel_c * hd, preferred_element_type=F32)

    P = _chain(ar, forward=True)
    Q = _chain(ar, forward=False)
    unr, uni, udr, udi = _prefix(P[K - 1], forward=True)
    wnr, wni, wdr, wdi = _prefix(Q[0], forward=False)

    g_rows_r, g_rows_i = [], []
    for k in range(K):
        (p00r, p00i, p01r, p01i, p10r, p10i, p11r, p11i) = P[k]
        nLr = p00r * unr - p00i * uni + p01r * udr - p01i * udi
        nLi = p00r * uni + p00i * unr + p01r * udi + p01i * udr
        dLr = p10r * unr - p10i * uni + p11r * udr - p11i * udi
        dLi = p10r * uni + p10i * unr + p11r * udi + p11i * udr
        dd = jnp.maximum(dLr * dLr + dLi * dLi, 1e-30)
        Lr = (nLr * dLr + nLi * dLi) / dd
        Li = (nLi * dLr - nLr * dLi) / dd
        (q00r, q00i, q01r, q01i, q10r, q10i, q11r, q11i) = Q[k]
        nRr = q00r * wnr - q00i * wni + q01r * wdr - q01i * wdi
        nRi = q00r * wni + q00i * wnr + q01r * wdi + q01i * wdr
        dRr = q10r * wnr - q10i * wni + q11r * wdr - q11i * wdi
        dRi = q10r * wni + q10i * wnr + q11r * wdi + q11i * wdr
        ddr = jnp.maximum(dRr * dRr + dRi * dRi, 1e-30)
        Rr = (nRr * dRr + nRi * dRi) / ddr
        Ri = (nRi * dRr - nRr * dRi) / ddr
        sr = Lr + Rr - ar[k:k + 1, :]
        si = Li + Ri + 1.0
        den = jnp.maximum(sr * sr + si * si, 1e-30)
        g_rows_r.append(jnp.clip(sr / den, -10.0, 10.0))
        g_rows_i.append(jnp.clip(-si / den, -10.0, 10.0))
    re_g = jnp.concatenate(g_rows_r, axis=0)
    im_g = jnp.concatenate(g_rows_i, axis=0)

    w1t = jnp.where((lax.broadcasted_iota(jnp.int32, (N, K), 0) % K)
                    == lax.broadcasted_iota(jnp.int32, (N, K), 1),
                    1.0, 0.0).astype(F32)
    col_r = jnp.sum(jnp.dot(w1t, re_g, preferred_element_type=F32) * sel_c,
                    axis=1, keepdims=True)
    col_i = jnp.sum(jnp.dot(w1t, im_g, preferred_element_type=F32) * sel_c,
                    axis=1, keepdims=True)
    spec = col_r * ow_ref[0:1, :] + col_i * ow_ref[1:2, :] + ob_ref[...]
    out_ref[...] = moe + sc_ref[0, 0] * spec


def _bk_final(yu, gate, pw, pb, ow, ob, bscale):
    return pl.pallas_call(
        _bk_body,
        grid=(1,),
        in_specs=[
            pl.BlockSpec((N, D), lambda i: (0, 0)),
            pl.BlockSpec((N, 1), lambda i: (0, 0)),
            pl.BlockSpec((D, 1), lambda i: (0, 0)),
            pl.BlockSpec((1, 1), lambda i: (0, 0)),
            pl.BlockSpec((2, D), lambda i: (0, 0)),
            pl.BlockSpec((1, D), lambda i: (0, 0)),
            pl.BlockSpec((1, 1), lambda i: (0, 0)),
        ],
        out_specs=pl.BlockSpec((N, D), lambda i: (0, 0)),
        out_shape=jax.ShapeDtypeStruct((N, D), F32),
    )(yu, gate, pw, pb, ow, ob, bscale)


def kernel(x, ln_gamma, ln_beta, router_w, router_b, w1, b1, w2, b2,
           pproj_w, pproj_b, oproj_w, oproj_b, bk_scale):
    x2 = x.reshape(N, D)
    g2, b2r = ln_gamma.reshape(1, D), ln_beta.reshape(1, D)
    gate, s, r_ids, e_ids, off = _ln_router(x2, g2, b2r, router_w,
                                            router_b.reshape(1, E))
    sf = s.reshape(N)
    xs = _sc_scatter(x2, sf)
    ys = _ffn_grouped(r_ids, e_ids, off, xs, g2, b2r, w1, b1, w2, b2)
    yu = _sc_gather(ys, sf)
    out = _bk_final(yu, gate, pproj_w, pproj_b.reshape(1, 1), oproj_w,
                    oproj_b.reshape(1, D), jnp.asarray(bk_scale).reshape(1, 1))
    return out.reshape(1, N, D)


# speedup vs baseline: 1.0586x; 1.0012x over previous
import functools

import jax
import jax.numpy as jnp
from jax import lax
from jax.experimental import pallas as pl
from jax.experimental.pallas import tpu as pltpu
from jax.experimental.pallas import tpu_sc as plsc

N, D, E, DFF = 2048, 768, 8, 3072
TB = 128
R = N // TB
W = R + E - 1
TBA = 256
K = 16
C = N // K
F32 = jnp.float32


_NB = N // TBA
_VS = 32


def _lnr_body(x_ref, g_ref, b_ref, rw_ref, rb_ref,
              gate_ref, s_ref, rid_ref, eid_ref, off_ref,
              acc_ref, idx_scr, rank_scr):
    i = pl.program_id(0)
    x = x_ref[...]
    mu = jnp.mean(x, axis=1, keepdims=True)
    var = jnp.mean((x - mu) ** 2, axis=1, keepdims=True)
    xn = (x - mu) / jnp.sqrt(var + 1e-5) * g_ref[...] + b_ref[...]
    logits = jnp.dot(xn, rw_ref[...], preferred_element_type=F32) + rb_ref[...]
    m = jnp.max(logits, axis=1, keepdims=True)
    gate_ref[...] = 1.0 / jnp.sum(jnp.exp(logits - m), axis=1, keepdims=True)
    e_iota = lax.broadcasted_iota(jnp.int32, (TBA, E), 1)
    idxv = jnp.min(jnp.where(logits >= m, e_iota, E), axis=1, keepdims=True)
    onehot = (idxv == e_iota).astype(F32)
    r1 = lax.broadcasted_iota(jnp.int32, (TBA, TBA), 0)
    c1 = lax.broadcasted_iota(jnp.int32, (TBA, TBA), 1)
    tri = (c1 <= r1).astype(F32)
    cum = jnp.dot(tri, onehot, preferred_element_type=F32)

    @pl.when(i == 0)
    def _():
        acc_ref[...] = jnp.zeros((1, E), F32)

    base = acc_ref[...]
    rank = jnp.sum(onehot * (cum + base), axis=1, keepdims=True) - 1.0
    idx_scr[pl.ds(i * TBA, TBA), :] = idxv.astype(jnp.int32)
    rank_scr[pl.ds(i * TBA, TBA), :] = rank
    acc_ref[...] = base + cum[TBA - 1:TBA, :]

    @pl.when(i == _NB - 1)
    def _():
        counts = acc_ref[...]
        ku = lax.broadcasted_iota(jnp.int32, (E, E), 0)
        eu = lax.broadcasted_iota(jnp.int32, (E, E), 1)
        upper = (ku < eu).astype(F32)
        off_ex = jnp.dot(counts, upper, preferred_element_type=F32,
                         precision=lax.Precision.HIGHEST)
        idx_all = idx_scr[...]
        oh_all = (idx_all == lax.broadcasted_iota(jnp.int32, (N, E), 1))
        s = rank_scr[...] + jnp.sum(oh_all.astype(F32) * off_ex, axis=1,
                                    keepdims=True)
        s_ref[...] = s.astype(jnp.int32)
        blocks = (lax.broadcasted_iota(jnp.int32, (1, R), 1) * TB).astype(F32)
        huge = jnp.full((1, _VS - (E - 1) - R), 1e9, F32)
        vals = jnp.concatenate([off_ex[:, 1:E], blocks, huge], axis=1)
        ident = (lax.broadcasted_iota(jnp.int32, (_VS, _VS), 0)
                 == lax.broadcasted_iota(jnp.int32, (_VS, _VS), 1)).astype(F32)
        vals_col = lax.dot_general(ident, vals, (((1,), (1,)), ((), ())),
                                   preferred_element_type=F32,
                                   precision=lax.Precision.HIGHEST)
        ii = lax.broadcasted_iota(jnp.int32, (_VS, _VS), 0)
        jj = lax.broadcasted_iota(jnp.int32, (_VS, _VS), 1)
        less = (vals < vals_col) | ((vals == vals_col) & (jj < ii))
        pos = jnp.sum(less.astype(F32), axis=1, keepdims=True)
        pmat = (pos == jj.astype(F32)).astype(F32)
        sortv = lax.dot_general(vals, pmat, (((1,), (0,)), ((), ())),
                                preferred_element_type=F32,
                                precision=lax.Precision.HIGHEST)
        rid = jnp.clip(jnp.floor(sortv * (1.0 / TB)), 0.0, R - 1.0)
        eacc = jnp.zeros((1, _VS), F32)
        for mth in range(E):
            eacc = eacc + (sortv >= off_ex[:, mth:mth + 1]).astype(F32)
        eacc = eacc + (sortv >= float(N)).astype(F32)
        rid_ref[...] = rid.astype(jnp.int32)
        eid_ref[...] = jnp.clip(eacc - 1.0, 0.0, E - 1.0).astype(jnp.int32)
        offp = jnp.concatenate(
            [off_ex, jnp.full((1, 1), float(N), F32),
             jnp.zeros((1, 16 - E - 1), F32)], axis=1)
        off_ref[...] = offp.astype(jnp.int32)


def _ln_router(x2, g, b, rw, rb):
    return pl.pallas_call(
        _lnr_body,
        grid=(_NB,),
        in_specs=[
            pl.BlockSpec((TBA, D), lambda i: (i, 0)),
            pl.BlockSpec((1, D), lambda i: (0, 0)),
            pl.BlockSpec((1, D), lambda i: (0, 0)),
            pl.BlockSpec((D, E), lambda i: (0, 0)),
            pl.BlockSpec((1, E), lambda i: (0, 0)),
        ],
        out_specs=[
            pl.BlockSpec((TBA, 1), lambda i: (i, 0)),
            pl.BlockSpec((N, 1), lambda i: (0, 0)),
            pl.BlockSpec((1, _VS), lambda i: (0, 0)),
            pl.BlockSpec((1, _VS), lambda i: (0, 0)),
            pl.BlockSpec((1, 16), lambda i: (0, 0)),
        ],
        out_shape=[
            jax.ShapeDtypeStruct((N, 1), F32),
            jax.ShapeDtypeStruct((N, 1), jnp.int32),
            jax.ShapeDtypeStruct((1, _VS), jnp.int32),
            jax.ShapeDtypeStruct((1, _VS), jnp.int32),
            jax.ShapeDtypeStruct((1, 16), jnp.int32),
        ],
        scratch_shapes=[pltpu.VMEM((1, E), F32),
                        pltpu.VMEM((N, 1), jnp.int32),
                        pltpu.VMEM((N, 1), F32)],
    )(x2, g, b, rw, rb)


_NC, _NS = 2, 16
_NW = _NC * _NS
_BPW = N // _NW


def _sc_gather(xn, perm):
    mesh = plsc.VectorSubcoreMesh(core_axis_name="c", subcore_axis_name="s")

    @functools.partial(
        pl.kernel, mesh=mesh,
        out_type=jax.ShapeDtypeStruct((N, D), F32),
        scratch_types=[
            pltpu.VMEM((_BPW,), jnp.int32),
            pltpu.VMEM((_BPW, D), F32),
            pltpu.SemaphoreType.DMA,
        ],
    )
    def k(xn_hbm, perm_hbm, out_hbm, idx_v, rows_v, sem):
        wid = lax.axis_index("s") * _NC + lax.axis_index("c")
        base = wid * _BPW
        pltpu.sync_copy(perm_hbm.at[pl.ds(base, _BPW)], idx_v)
        pltpu.async_copy(xn_hbm.at[idx_v], rows_v, sem).wait()
        pltpu.sync_copy(rows_v, out_hbm.at[pl.ds(base, _BPW)])

    return k(xn, perm)


def _sc_scatter(ys, perm):
    mesh = plsc.VectorSubcoreMesh(core_axis_name="c", subcore_axis_name="s")

    @functools.partial(
        pl.kernel, mesh=mesh,
        out_type=jax.ShapeDtypeStruct((N, D), F32),
        scratch_types=[
            pltpu.VMEM((_BPW,), jnp.int32),
            pltpu.VMEM((_BPW, D), F32),
            pltpu.SemaphoreType.DMA,
        ],
    )
    def k(ys_hbm, perm_hbm, out_hbm, idx_v, rows_v, sem):
        wid = lax.axis_index("s") * _NC + lax.axis_index("c")
        base = wid * _BPW
        pltpu.sync_copy(perm_hbm.at[pl.ds(base, _BPW)], idx_v)
        pltpu.sync_copy(ys_hbm.at[pl.ds(base, _BPW)], rows_v)
        pltpu.async_copy(rows_v, out_hbm.at[idx_v], sem).wait()

    return k(ys, perm)


def _ffn_body(r_ref, e_ref, off_ref,
              xs_ref, g_ref, b_ref, w1_ref, b1_ref, w2_ref, b2_ref, out_ref):
    j = pl.program_id(0)
    r = r_ref[0, j]
    e = e_ref[0, j]
    x = xs_ref[...]
    mu = jnp.mean(x, axis=1, keepdims=True)
    var = jnp.mean((x - mu) ** 2, axis=1, keepdims=True)
    xn = (x - mu) / jnp.sqrt(var + 1e-5) * g_ref[...] + b_ref[...]
    h = jnp.dot(xn, w1_ref[0], preferred_element_type=F32) + b1_ref[0]
    h = jax.nn.gelu(h)
    y = jnp.dot(h, w2_ref[0], preferred_element_type=F32) + b2_ref[0]
    jp = jnp.maximum(j - 1, 0)
    dup = (j > 0) & (r == r_ref[0, jp]) & (e == e_ref[0, jp])
    lo = jnp.maximum(off_ref[0, e], r * TB)
    hi = jnp.where(dup, lo, jnp.minimum(off_ref[0, e + 1], (r + 1) * TB))
    rows = r * TB + lax.broadcasted_iota(jnp.int32, (TB, 1), 0)
    mask = (rows >= lo) & (rows < hi)
    contrib = jnp.where(mask, y, 0.0)
    first = jnp.logical_or(j == 0, r != r_ref[0, jp])

    @pl.when(first)
    def _():
        out_ref[...] = contrib

    @pl.when(jnp.logical_not(first))
    def _():
        out_ref[...] += contrib


def _ffn_grouped(r_ids, e_ids, off, xs, g, b, w1, b1, w2, b2):
    grid_spec = pltpu.PrefetchScalarGridSpec(
        num_scalar_prefetch=3,
        grid=(W,),
        in_specs=[
            pl.BlockSpec((TB, D), lambda j, r, e, off: (r[0, j], 0)),
            pl.BlockSpec((1, D), lambda j, r, e, off: (0, 0)),
            pl.BlockSpec((1, D), lambda j, r, e, off: (0, 0)),
            pl.BlockSpec((1, D, DFF), lambda j, r, e, off: (e[0, j], 0, 0)),
            pl.BlockSpec((1, 1, DFF), lambda j, r, e, off: (e[0, j], 0, 0)),
            pl.BlockSpec((1, DFF, D), lambda j, r, e, off: (e[0, j], 0, 0)),
            pl.BlockSpec((1, 1, D), lambda j, r, e, off: (e[0, j], 0, 0)),
        ],
        out_specs=pl.BlockSpec((TB, D), lambda j, r, e, off: (r[0, j], 0)),
    )
    return pl.pallas_call(
        _ffn_body,
        grid_spec=grid_spec,
        out_shape=jax.ShapeDtypeStruct((N, D), F32),
    )(r_ids, e_ids, off, xs, g, b, w1, b1.reshape(E, 1, DFF), w2,
      b2.reshape(E, 1, D))


_ID8 = (1.0, 0.0, 0.0, 0.0, 0.0, 0.0, 1.0, 0.0)


def _mm2x2(a, b):
    (a00r, a00i, a01r, a01i, a10r, a10i, a11r, a11i) = a
    (b00r, b00i, b01r, b01i, b10r, b10i, b11r, b11i) = b
    c00r = a00r * b00r - a00i * b00i + a01r * b10r - a01i * b10i
    c00i = a00r * b00i + a00i * b00r + a01r * b10i + a01i * b10r
    c01r = a00r * b01r - a00i * b01i + a01r * b11r - a01i * b11i
    c01i = a00r * b01i + a00i * b01r + a01r * b11i + a01i * b11r
    c10r = a10r * b00r - a10i * b00i + a11r * b10r - a11i * b10i
    c10i = a10r * b00i + a10i * b00r + a11r * b10i + a11i * b10r
    c11r = a10r * b01r - a10i * b01i + a11r * b11r - a11i * b11i
    c11i = a10r * b01i + a10i * b01r + a11r * b11i + a11i * b11r
    return (c00r, c00i, c01r, c01i, c10r, c10i, c11r, c11i)


def _mnorm(m):
    mx = m[0] * 0.0
    for t in m:
        mx = jnp.maximum(mx, jnp.abs(t))
    s = 1.0 / jnp.maximum(mx, 1e-30)
    return tuple(t * s for t in m)


def _shift(m, s, right):
    stacked = jnp.concatenate(m, axis=0)
    ii = lax.broadcasted_iota(jnp.int32, (C, C), 0)
    jj = lax.broadcasted_iota(jnp.int32, (C, C), 1)
    d = s if right else -s
    sel = jnp.where(jj - ii == d, 1.0, 0.0).astype(F32)
    sh = jnp.dot(stacked, sel, preferred_element_type=F32)
    lanes = lax.broadcasted_iota(jnp.int32, (1, C), 1)
    cond = (lanes < s) if right else (lanes >= C - s)
    return tuple(jnp.where(cond, idv, sh[t:t + 1])
                 for t, idv in enumerate(_ID8))


def _chain(ar, forward):
    one = jnp.ones((1, C), F32)
    zero = jnp.zeros((1, C), F32)
    order = range(K) if forward else range(K - 1, -1, -1)
    out = [None] * K
    p = None
    for k in order:
        arr = ar[k:k + 1, :]
        if p is None:
            p = (arr, -one, -one, zero, one, zero, zero, zero)
        else:
            (p00r, p00i, p01r, p01i, p10r, p10i, p11r, p11i) = p
            n00r = arr * p00r + p00i - p10r
            n00i = arr * p00i - p00r - p10i
            n01r = arr * p01r + p01i - p11r
            n01i = arr * p01i - p01r - p11i
            p = (n00r, n00i, n01r, n01i, p00r, p00i, p01r, p01i)
        out[k] = p
    return out


def _prefix(m0, forward):
    x = _mnorm(m0)
    s = 1
    while s < C:
        xs = _shift(x, s, right=forward)
        x = _mnorm(_mm2x2(x, xs))
        s *= 2
    xs = _shift(x, 1, right=forward)
    return xs[0], xs[1], xs[4], xs[5]


def _bk_body(ys_ref, gate_ref, pw_ref, pb_ref, ow_ref, ob_ref, sc_ref, out_ref):
    ys = ys_ref[...]
    moe = ys * gate_ref[...]
    v = jnp.dot(moe, pw_ref[...], preferred_element_type=F32) + pb_ref[0, 0]
    hd = jnp.clip(v, -3.0, 3.0) - 2.0
    i2 = lax.broadcasted_iota(jnp.int32, (N, C), 0)
    c2 = lax.broadcasted_iota(jnp.int32, (N, C), 1)
    sel_c = jnp.where((i2 // K) == c2, 1.0, 0.0).astype(F32)
    k16 = lax.broadcasted_iota(jnp.int32, (K, N), 0)
    i16 = lax.broadcasted_iota(jnp.int32, (K, N), 1)
    w1sel = jnp.where((i16 % K) == k16, 1.0, 0.0).astype(F32)
    ar = jnp.dot(w1sel, sel_c * hd, preferred_element_type=F32)

    P = _chain(ar, forward=True)
    Q = _chain(ar, forward=False)
    unr, uni, udr, udi = _prefix(P[K - 1], forward=True)
    wnr, wni, wdr, wdi = _prefix(Q[0], forward=False)

    g_rows_r, g_rows_i = [], []
    for k in range(K):
        (p00r, p00i, p01r, p01i, p10r, p10i, p11r, p11i) = P[k]
        nLr = p00r * unr - p00i * uni + p01r * udr - p01i * udi
        nLi = p00r * uni + p00i * unr + p01r * udi + p01i * udr
        dLr = p10r * unr - p10i * uni + p11r * udr - p11i * udi
        dLi = p10r * uni + p10i * unr + p11r * udi + p11i * udr
        dd = jnp.maximum(dLr * dLr + dLi * dLi, 1e-30)
        Lr = (nLr * dLr + nLi * dLi) / dd
        Li = (nLi * dLr - nLr * dLi) / dd
        (q00r, q00i, q01r, q01i, q10r, q10i, q11r, q11i) = Q[k]
        nRr = q00r * wnr - q00i * wni + q01r * wdr - q01i * wdi
        nRi = q00r * wni + q00i * wnr + q01r * wdi + q01i * wdr
        dRr = q10r * wnr - q10i * wni + q11r * wdr - q11i * wdi
        dRi = q10r * wni + q10i * wnr + q11r * wdi + q11i * wdr
        ddr = jnp.maximum(dRr * dRr + dRi * dRi, 1e-30)
        Rr = (nRr * dRr + nRi * dRi) / ddr
        Ri = (nRi * dRr - nRr * dRi) / ddr
        sr = Lr + Rr - ar[k:k + 1, :]
        si = Li + Ri + 1.0
        den = jnp.maximum(sr * sr + si * si, 1e-30)
        g_rows_r.append(jnp.clip(sr / den, -10.0, 10.0))
        g_rows_i.append(jnp.clip(-si / den, -10.0, 10.0))
    re_g = jnp.concatenate(g_rows_r, axis=0)
    im_g = jnp.concatenate(g_rows_i, axis=0)

    w1t = jnp.where((lax.broadcasted_iota(jnp.int32, (N, K), 0) % K)
                    == lax.broadcasted_iota(jnp.int32, (N, K), 1),
                    1.0, 0.0).astype(F32)
    col_r = jnp.sum(jnp.dot(w1t, re_g, preferred_element_type=F32) * sel_c,
                    axis=1, keepdims=True)
    col_i = jnp.sum(jnp.dot(w1t, im_g, preferred_element_type=F32) * sel_c,
                    axis=1, keepdims=True)
    spec = col_r * ow_ref[0:1, :] + col_i * ow_ref[1:2, :] + ob_ref[...]
    out_ref[...] = moe + sc_ref[0, 0] * spec


def _bk_final(yu, gate, pw, pb, ow, ob, bscale):
    return pl.pallas_call(
        _bk_body,
        grid=(1,),
        in_specs=[
            pl.BlockSpec((N, D), lambda i: (0, 0)),
            pl.BlockSpec((N, 1), lambda i: (0, 0)),
            pl.BlockSpec((D, 1), lambda i: (0, 0)),
            pl.BlockSpec((1, 1), lambda i: (0, 0)),
            pl.BlockSpec((2, D), lambda i: (0, 0)),
            pl.BlockSpec((1, D), lambda i: (0, 0)),
            pl.BlockSpec((1, 1), lambda i: (0, 0)),
        ],
        out_specs=pl.BlockSpec((N, D), lambda i: (0, 0)),
        out_shape=jax.ShapeDtypeStruct((N, D), F32),
    )(yu, gate, pw, pb, ow, ob, bscale)


def kernel(x, ln_gamma, ln_beta, router_w, router_b, w1, b1, w2, b2,
           pproj_w, pproj_b, oproj_w, oproj_b, bk_scale):
    x2 = x.reshape(N, D)
    g2, b2r = ln_gamma.reshape(1, D), ln_beta.reshape(1, D)
    gate, s, r_ids, e_ids, off = _ln_router(x2, g2, b2r, router_w,
                                            router_b.reshape(1, E))
    sf = s.reshape(N)
    xs = _sc_scatter(x2, sf)
    ys = _ffn_grouped(r_ids, e_ids, off, xs, g2, b2r, w1, b1, w2, b2)
    yu = _sc_gather(ys, sf)
    out = _bk_final(yu, gate, pproj_w, pproj_b.reshape(1, 1), oproj_w,
                    oproj_b.reshape(1, D), jnp.asarray(bk_scale).reshape(1, 1))
    return out.reshape(1, N, D)


# speedup vs baseline: 1.0605x; 1.0018x over previous
import functools

import jax
import jax.numpy as jnp
from jax import lax
from jax.experimental import pallas as pl
from jax.experimental.pallas import tpu as pltpu
from jax.experimental.pallas import tpu_sc as plsc

N, D, E, DFF = 2048, 768, 8, 3072
TB = 128
R = N // TB
W = R + E - 1
TBA = 256
K = 16
C = N // K
F32 = jnp.float32


_NB = N // TBA
_VS = 32


def _lnr_body(x_ref, g_ref, b_ref, rw_ref, rb_ref,
              gate_ref, s_ref, rid_ref, eid_ref, off_ref,
              acc_ref, idx_scr, rank_scr):
    i = pl.program_id(0)
    x = x_ref[...]
    mu = jnp.mean(x, axis=1, keepdims=True)
    var = jnp.mean((x - mu) ** 2, axis=1, keepdims=True)
    xn = (x - mu) / jnp.sqrt(var + 1e-5) * g_ref[...] + b_ref[...]
    logits = jnp.dot(xn, rw_ref[...], preferred_element_type=F32) + rb_ref[...]
    m = jnp.max(logits, axis=1, keepdims=True)
    gate_ref[...] = 1.0 / jnp.sum(jnp.exp(logits - m), axis=1, keepdims=True)
    e_iota = lax.broadcasted_iota(jnp.int32, (TBA, E), 1)
    idxv = jnp.min(jnp.where(logits >= m, e_iota, E), axis=1, keepdims=True)
    onehot = (idxv == e_iota).astype(F32)
    r1 = lax.broadcasted_iota(jnp.int32, (TBA, TBA), 0)
    c1 = lax.broadcasted_iota(jnp.int32, (TBA, TBA), 1)
    tri = (c1 <= r1).astype(F32)
    cum = jnp.dot(tri, onehot, preferred_element_type=F32)

    @pl.when(i == 0)
    def _():
        acc_ref[...] = jnp.zeros((1, E), F32)

    base = acc_ref[...]
    rank = jnp.sum(onehot * (cum + base), axis=1, keepdims=True) - 1.0
    idx_scr[pl.ds(i * TBA, TBA), :] = idxv.astype(jnp.int32)
    rank_scr[pl.ds(i * TBA, TBA), :] = rank
    acc_ref[...] = base + cum[TBA - 1:TBA, :]

    @pl.when(i == _NB - 1)
    def _():
        counts = acc_ref[...]
        ku = lax.broadcasted_iota(jnp.int32, (E, E), 0)
        eu = lax.broadcasted_iota(jnp.int32, (E, E), 1)
        upper = (ku < eu).astype(F32)
        off_ex = jnp.dot(counts, upper, preferred_element_type=F32,
                         precision=lax.Precision.HIGHEST)
        idx_all = idx_scr[...]
        oh_all = (idx_all == lax.broadcasted_iota(jnp.int32, (N, E), 1))
        s = rank_scr[...] + jnp.sum(oh_all.astype(F32) * off_ex, axis=1,
                                    keepdims=True)
        s_ref[...] = s.astype(jnp.int32)
        blocks = (lax.broadcasted_iota(jnp.int32, (1, R), 1) * TB).astype(F32)
        huge = jnp.full((1, _VS - (E - 1) - R), 1e9, F32)
        vals = jnp.concatenate([off_ex[:, 1:E], blocks, huge], axis=1)
        ident = (lax.broadcasted_iota(jnp.int32, (_VS, _VS), 0)
                 == lax.broadcasted_iota(jnp.int32, (_VS, _VS), 1)).astype(F32)
        vals_col = lax.dot_general(ident, vals, (((1,), (1,)), ((), ())),
                                   preferred_element_type=F32,
                                   precision=lax.Precision.HIGHEST)
        ii = lax.broadcasted_iota(jnp.int32, (_VS, _VS), 0)
        jj = lax.broadcasted_iota(jnp.int32, (_VS, _VS), 1)
        less = (vals < vals_col) | ((vals == vals_col) & (jj < ii))
        pos = jnp.sum(less.astype(F32), axis=1, keepdims=True)
        pmat = (pos == jj.astype(F32)).astype(F32)
        sortv = lax.dot_general(vals, pmat, (((1,), (0,)), ((), ())),
                                preferred_element_type=F32,
                                precision=lax.Precision.HIGHEST)
        rid = jnp.clip(jnp.floor(sortv * (1.0 / TB)), 0.0, R - 1.0)
        eacc = jnp.zeros((1, _VS), F32)
        for mth in range(E):
            eacc = eacc + (sortv >= off_ex[:, mth:mth + 1]).astype(F32)
        eacc = eacc + (sortv >= float(N)).astype(F32)
        rid_ref[...] = rid.astype(jnp.int32)
        eid_ref[...] = jnp.clip(eacc - 1.0, 0.0, E - 1.0).astype(jnp.int32)
        offp = jnp.concatenate(
            [off_ex, jnp.full((1, 1), float(N), F32),
             jnp.zeros((1, 16 - E - 1), F32)], axis=1)
        off_ref[...] = offp.astype(jnp.int32)


def _ln_router(x2, g, b, rw, rb):
    return pl.pallas_call(
        _lnr_body,
        grid=(_NB,),
        in_specs=[
            pl.BlockSpec((TBA, D), lambda i: (i, 0)),
            pl.BlockSpec((1, D), lambda i: (0, 0)),
            pl.BlockSpec((1, D), lambda i: (0, 0)),
            pl.BlockSpec((D, E), lambda i: (0, 0)),
            pl.BlockSpec((1, E), lambda i: (0, 0)),
        ],
        out_specs=[
            pl.BlockSpec((TBA, 1), lambda i: (i, 0)),
            pl.BlockSpec((N, 1), lambda i: (0, 0)),
            pl.BlockSpec((1, _VS), lambda i: (0, 0)),
            pl.BlockSpec((1, _VS), lambda i: (0, 0)),
            pl.BlockSpec((1, 16), lambda i: (0, 0)),
        ],
        out_shape=[
            jax.ShapeDtypeStruct((N, 1), F32),
            jax.ShapeDtypeStruct((N, 1), jnp.int32),
            jax.ShapeDtypeStruct((1, _VS), jnp.int32),
            jax.ShapeDtypeStruct((1, _VS), jnp.int32),
            jax.ShapeDtypeStruct((1, 16), jnp.int32),
        ],
        scratch_shapes=[pltpu.VMEM((1, E), F32),
                        pltpu.VMEM((N, 1), jnp.int32),
                        pltpu.VMEM((N, 1), F32)],
    )(x2, g, b, rw, rb)


_NC, _NS = 2, 16
_NW = _NC * _NS
_BPW = N // _NW


def _sc_gather(xn, perm):
    mesh = plsc.VectorSubcoreMesh(core_axis_name="c", subcore_axis_name="s")

    @functools.partial(
        pl.kernel, mesh=mesh,
        out_type=jax.ShapeDtypeStruct((N, D), F32),
        scratch_types=[
            pltpu.VMEM((_BPW,), jnp.int32),
            pltpu.VMEM((_BPW, D), F32),
            pltpu.SemaphoreType.DMA,
        ],
    )
    def k(xn_hbm, perm_hbm, out_hbm, idx_v, rows_v, sem):
        wid = lax.axis_index("s") * _NC + lax.axis_index("c")
        base = wid * _BPW
        pltpu.sync_copy(perm_hbm.at[pl.ds(base, _BPW)], idx_v)
        pltpu.async_copy(xn_hbm.at[idx_v], rows_v, sem).wait()
        pltpu.sync_copy(rows_v, out_hbm.at[pl.ds(base, _BPW)])

    return k(xn, perm)


def _sc_scatter(ys, perm):
    mesh = plsc.VectorSubcoreMesh(core_axis_name="c", subcore_axis_name="s")

    @functools.partial(
        pl.kernel, mesh=mesh,
        out_type=jax.ShapeDtypeStruct((N, D), F32),
        scratch_types=[
            pltpu.VMEM((_BPW,), jnp.int32),
            pltpu.VMEM((_BPW, D), F32),
            pltpu.SemaphoreType.DMA,
        ],
    )
    def k(ys_hbm, perm_hbm, out_hbm, idx_v, rows_v, sem):
        wid = lax.axis_index("s") * _NC + lax.axis_index("c")
        base = wid * _BPW
        pltpu.sync_copy(perm_hbm.at[pl.ds(base, _BPW)], idx_v)
        pltpu.sync_copy(ys_hbm.at[pl.ds(base, _BPW)], rows_v)
        pltpu.async_copy(rows_v, out_hbm.at[idx_v], sem).wait()

    return k(ys, perm)


def _ffn_body(r_ref, e_ref, off_ref,
              xs_ref, g_ref, b_ref, w1_ref, b1_ref, w2_ref, b2_ref, out_ref):
    j = pl.program_id(0)
    r = r_ref[0, j]
    e = e_ref[0, j]
    x = xs_ref[...]
    mu = jnp.mean(x, axis=1, keepdims=True)
    var = jnp.mean((x - mu) ** 2, axis=1, keepdims=True)
    xn = (x - mu) / jnp.sqrt(var + 1e-5) * g_ref[...] + b_ref[...]
    h = jnp.dot(xn, w1_ref[0], preferred_element_type=F32) + b1_ref[pl.ds(e, 1)]
    h = jax.nn.gelu(h)
    y = jnp.dot(h, w2_ref[0], preferred_element_type=F32) + b2_ref[pl.ds(e, 1)]
    jp = jnp.maximum(j - 1, 0)
    dup = (j > 0) & (r == r_ref[0, jp]) & (e == e_ref[0, jp])
    lo = jnp.maximum(off_ref[0, e], r * TB)
    hi = jnp.where(dup, lo, jnp.minimum(off_ref[0, e + 1], (r + 1) * TB))
    rows = r * TB + lax.broadcasted_iota(jnp.int32, (TB, 1), 0)
    mask = (rows >= lo) & (rows < hi)
    contrib = jnp.where(mask, y, 0.0)
    first = jnp.logical_or(j == 0, r != r_ref[0, jp])

    @pl.when(first)
    def _():
        out_ref[...] = contrib

    @pl.when(jnp.logical_not(first))
    def _():
        out_ref[...] += contrib


def _ffn_grouped(r_ids, e_ids, off, xs, g, b, w1, b1, w2, b2):
    grid_spec = pltpu.PrefetchScalarGridSpec(
        num_scalar_prefetch=3,
        grid=(W,),
        in_specs=[
            pl.BlockSpec((TB, D), lambda j, r, e, off: (r[0, j], 0)),
            pl.BlockSpec((1, D), lambda j, r, e, off: (0, 0)),
            pl.BlockSpec((1, D), lambda j, r, e, off: (0, 0)),
            pl.BlockSpec((1, D, DFF), lambda j, r, e, off: (e[0, j], 0, 0)),
            pl.BlockSpec((E, DFF), lambda j, r, e, off: (0, 0)),
            pl.BlockSpec((1, DFF, D), lambda j, r, e, off: (e[0, j], 0, 0)),
            pl.BlockSpec((E, D), lambda j, r, e, off: (0, 0)),
        ],
        out_specs=pl.BlockSpec((TB, D), lambda j, r, e, off: (r[0, j], 0)),
    )
    return pl.pallas_call(
        _ffn_body,
        grid_spec=grid_spec,
        out_shape=jax.ShapeDtypeStruct((N, D), F32),
    )(r_ids, e_ids, off, xs, g, b, w1, b1, w2, b2)


_ID8 = (1.0, 0.0, 0.0, 0.0, 0.0, 0.0, 1.0, 0.0)


def _mm2x2(a, b):
    (a00r, a00i, a01r, a01i, a10r, a10i, a11r, a11i) = a
    (b00r, b00i, b01r, b01i, b10r, b10i, b11r, b11i) = b
    c00r = a00r * b00r - a00i * b00i + a01r * b10r - a01i * b10i
    c00i = a00r * b00i + a00i * b00r + a01r * b10i + a01i * b10r
    c01r = a00r * b01r - a00i * b01i + a01r * b11r - a01i * b11i
    c01i = a00r * b01i + a00i * b01r + a01r * b11i + a01i * b11r
    c10r = a10r * b00r - a10i * b00i + a11r * b10r - a11i * b10i
    c10i = a10r * b00i + a10i * b00r + a11r * b10i + a11i * b10r
    c11r = a10r * b01r - a10i * b01i + a11r * b11r - a11i * b11i
    c11i = a10r * b01i + a10i * b01r + a11r * b11i + a11i * b11r
    return (c00r, c00i, c01r, c01i, c10r, c10i, c11r, c11i)


def _mnorm(m):
    mx = m[0] * 0.0
    for t in m:
        mx = jnp.maximum(mx, jnp.abs(t))
    s = 1.0 / jnp.maximum(mx, 1e-30)
    return tuple(t * s for t in m)


def _shift(m, s, right):
    stacked = jnp.concatenate(m, axis=0)
    ii = lax.broadcasted_iota(jnp.int32, (C, C), 0)
    jj = lax.broadcasted_iota(jnp.int32, (C, C), 1)
    d = s if right else -s
    sel = jnp.where(jj - ii == d, 1.0, 0.0).astype(F32)
    sh = jnp.dot(stacked, sel, preferred_element_type=F32)
    lanes = lax.broadcasted_iota(jnp.int32, (1, C), 1)
    cond = (lanes < s) if right else (lanes >= C - s)
    return tuple(jnp.where(cond, idv, sh[t:t + 1])
                 for t, idv in enumerate(_ID8))


def _chain(ar, forward):
    one = jnp.ones((1, C), F32)
    zero = jnp.zeros((1, C), F32)
    order = range(K) if forward else range(K - 1, -1, -1)
    out = [None] * K
    p = None
    for k in order:
        arr = ar[k:k + 1, :]
        if p is None:
            p = (arr, -one, -one, zero, one, zero, zero, zero)
        else:
            (p00r, p00i, p01r, p01i, p10r, p10i, p11r, p11i) = p
            n00r = arr * p00r + p00i - p10r
            n00i = arr * p00i - p00r - p10i
            n01r = arr * p01r + p01i - p11r
            n01i = arr * p01i - p01r - p11i
            p = (n00r, n00i, n01r, n01i, p00r, p00i, p01r, p01i)
        out[k] = p
    return out


def _prefix(m0, forward):
    x = _mnorm(m0)
    s = 1
    while s < C:
        xs = _shift(x, s, right=forward)
        x = _mnorm(_mm2x2(x, xs))
        s *= 2
    xs = _shift(x, 1, right=forward)
    return xs[0], xs[1], xs[4], xs[5]


def _bk_body(ys_ref, gate_ref, pw_ref, pb_ref, ow_ref, ob_ref, sc_ref, out_ref):
    ys = ys_ref[...]
    moe = ys * gate_ref[...]
    v = jnp.dot(moe, pw_ref[...], preferred_element_type=F32) + pb_ref[0, 0]
    hd = jnp.clip(v, -3.0, 3.0) - 2.0
    i2 = lax.broadcasted_iota(jnp.int32, (N, C), 0)
    c2 = lax.broadcasted_iota(jnp.int32, (N, C), 1)
    sel_c = jnp.where((i2 // K) == c2, 1.0, 0.0).astype(F32)
    k16 = lax.broadcasted_iota(jnp.int32, (K, N), 0)
    i16 = lax.broadcasted_iota(jnp.int32, (K, N), 1)
    w1sel = jnp.where((i16 % K) == k16, 1.0, 0.0).astype(F32)
    ar = jnp.dot(w1sel, sel_c * hd, preferred_element_type=F32)

    P = _chain(ar, forward=True)
    Q = _chain(ar, forward=False)
    unr, uni, udr, udi = _prefix(P[K - 1], forward=True)
    wnr, wni, wdr, wdi = _prefix(Q[0], forward=False)

    g_rows_r, g_rows_i = [], []
    for k in range(K):
        (p00r, p00i, p01r, p01i, p10r, p10i, p11r, p11i) = P[k]
        nLr = p00r * unr - p00i * uni + p01r * udr - p01i * udi
        nLi = p00r * uni + p00i * unr + p01r * udi + p01i * udr
        dLr = p10r * unr - p10i * uni + p11r * udr - p11i * udi
        dLi = p10r * uni + p10i * unr + p11r * udi + p11i * udr
        dd = jnp.maximum(dLr * dLr + dLi * dLi, 1e-30)
        Lr = (nLr * dLr + nLi * dLi) / dd
        Li = (nLi * dLr - nLr * dLi) / dd
        (q00r, q00i, q01r, q01i, q10r, q10i, q11r, q11i) = Q[k]
        nRr = q00r * wnr - q00i * wni + q01r * wdr - q01i * wdi
        nRi = q00r * wni + q00i * wnr + q01r * wdi + q01i * wdr
        dRr = q10r * wnr - q10i * wni + q11r * wdr - q11i * wdi
        dRi = q10r * wni + q10i * wnr + q11r * wdi + q11i * wdr
        ddr = jnp.maximum(dRr * dRr + dRi * dRi, 1e-30)
        Rr = (nRr * dRr + nRi * dRi) / ddr
        Ri = (nRi * dRr - nRr * dRi) / ddr
        sr = Lr + Rr - ar[k:k + 1, :]
        si = Li + Ri + 1.0
        den = jnp.maximum(sr * sr + si * si, 1e-30)
        g_rows_r.append(jnp.clip(sr / den, -10.0, 10.0))
        g_rows_i.append(jnp.clip(-si / den, -10.0, 10.0))
    re_g = jnp.concatenate(g_rows_r, axis=0)
    im_g = jnp.concatenate(g_rows_i, axis=0)

    w1t = jnp.where((lax.broadcasted_iota(jnp.int32, (N, K), 0) % K)
                    == lax.broadcasted_iota(jnp.int32, (N, K), 1),
                    1.0, 0.0).astype(F32)
    col_r = jnp.sum(jnp.dot(w1t, re_g, preferred_element_type=F32) * sel_c,
                    axis=1, keepdims=True)
    col_i = jnp.sum(jnp.dot(w1t, im_g, preferred_element_type=F32) * sel_c,
                    axis=1, keepdims=True)
    spec = col_r * ow_ref[0:1, :] + col_i * ow_ref[1:2, :] + ob_ref[...]
    out_ref[...] = (moe + sc_ref[0, 0] * spec)[None]


def _bk_final(yu, gate, pw, pb, ow, ob, bscale):
    return pl.pallas_call(
        _bk_body,
        grid=(1,),
        in_specs=[
            pl.BlockSpec((N, D), lambda i: (0, 0)),
            pl.BlockSpec((N, 1), lambda i: (0, 0)),
            pl.BlockSpec((D, 1), lambda i: (0, 0)),
            pl.BlockSpec((1, 1), lambda i: (0, 0)),
            pl.BlockSpec((2, D), lambda i: (0, 0)),
            pl.BlockSpec((1, D), lambda i: (0, 0)),
            pl.BlockSpec((1, 1), lambda i: (0, 0)),
        ],
        out_specs=pl.BlockSpec((1, N, D), lambda i: (0, 0, 0)),
        out_shape=jax.ShapeDtypeStruct((1, N, D), F32),
    )(yu, gate, pw, pb, ow, ob, bscale)


def kernel(x, ln_gamma, ln_beta, router_w, router_b, w1, b1, w2, b2,
           pproj_w, pproj_b, oproj_w, oproj_b, bk_scale):
    x2 = x.reshape(N, D)
    g2, b2r = ln_gamma.reshape(1, D), ln_beta.reshape(1, D)
    gate, s, r_ids, e_ids, off = _ln_router(x2, g2, b2r, router_w,
                                            router_b.reshape(1, E))
    sf = s.reshape(N)
    xs = _sc_scatter(x2, sf)
    ys = _ffn_grouped(r_ids, e_ids, off, xs, g2, b2r, w1, b1, w2, b2)
    yu = _sc_gather(ys, sf)
    return _bk_final(yu, gate, pproj_w, pproj_b.reshape(1, 1), oproj_w,
                     oproj_b.reshape(1, D), jnp.asarray(bk_scale).reshape(1, 1))


# speedup vs baseline: 1.0859x; 1.0240x over previous
import functools

import jax
import jax.numpy as jnp
from jax import lax
from jax.experimental import pallas as pl
from jax.experimental.pallas import tpu as pltpu
from jax.experimental.pallas import tpu_sc as plsc

N, D, E, DFF = 2048, 768, 8, 3072
TB = 128
R = N // TB
W = R + E - 1
TBA = 512
K = 16
C = N // K
F32 = jnp.float32


_NB = N // TBA
_VS = 32


def _lnr_body(x_ref, g_ref, b_ref, rw_ref, rb_ref,
              gate_ref, s_ref, rid_ref, eid_ref, off_ref,
              acc_ref, idx_scr, rank_scr):
    i = pl.program_id(0)
    x = x_ref[...]
    mu = jnp.mean(x, axis=1, keepdims=True)
    var = jnp.mean((x - mu) ** 2, axis=1, keepdims=True)
    xn = (x - mu) / jnp.sqrt(var + 1e-5) * g_ref[...] + b_ref[...]
    logits = jnp.dot(xn, rw_ref[...], preferred_element_type=F32) + rb_ref[...]
    m = jnp.max(logits, axis=1, keepdims=True)
    gate_ref[...] = 1.0 / jnp.sum(jnp.exp(logits - m), axis=1, keepdims=True)
    e_iota = lax.broadcasted_iota(jnp.int32, (TBA, E), 1)
    idxv = jnp.min(jnp.where(logits >= m, e_iota, E), axis=1, keepdims=True)
    onehot = (idxv == e_iota).astype(F32)
    r1 = lax.broadcasted_iota(jnp.int32, (TBA, TBA), 0)
    c1 = lax.broadcasted_iota(jnp.int32, (TBA, TBA), 1)
    tri = (c1 <= r1).astype(F32)
    cum = jnp.dot(tri, onehot, preferred_element_type=F32)

    @pl.when(i == 0)
    def _():
        acc_ref[...] = jnp.zeros((1, E), F32)

    base = acc_ref[...]
    rank = jnp.sum(onehot * (cum + base), axis=1, keepdims=True) - 1.0
    idx_scr[pl.ds(i * TBA, TBA), :] = idxv.astype(jnp.int32)
    rank_scr[pl.ds(i * TBA, TBA), :] = rank
    acc_ref[...] = base + cum[TBA - 1:TBA, :]

    @pl.when(i == _NB - 1)
    def _():
        counts = acc_ref[...]
        ku = lax.broadcasted_iota(jnp.int32, (E, E), 0)
        eu = lax.broadcasted_iota(jnp.int32, (E, E), 1)
        upper = (ku < eu).astype(F32)
        off_ex = jnp.dot(counts, upper, preferred_element_type=F32,
                         precision=lax.Precision.HIGHEST)
        idx_all = idx_scr[...]
        oh_all = (idx_all == lax.broadcasted_iota(jnp.int32, (N, E), 1))
        s = rank_scr[...] + jnp.sum(oh_all.astype(F32) * off_ex, axis=1,
                                    keepdims=True)
        s_ref[...] = s.astype(jnp.int32)
        blocks = (lax.broadcasted_iota(jnp.int32, (1, R), 1) * TB).astype(F32)
        huge = jnp.full((1, _VS - (E - 1) - R), 1e9, F32)
        vals = jnp.concatenate([off_ex[:, 1:E], blocks, huge], axis=1)
        ident = (lax.broadcasted_iota(jnp.int32, (_VS, _VS), 0)
                 == lax.broadcasted_iota(jnp.int32, (_VS, _VS), 1)).astype(F32)
        vals_col = lax.dot_general(ident, vals, (((1,), (1,)), ((), ())),
                                   preferred_element_type=F32,
                                   precision=lax.Precision.HIGHEST)
        ii = lax.broadcasted_iota(jnp.int32, (_VS, _VS), 0)
        jj = lax.broadcasted_iota(jnp.int32, (_VS, _VS), 1)
        less = (vals < vals_col) | ((vals == vals_col) & (jj < ii))
        pos = jnp.sum(less.astype(F32), axis=1, keepdims=True)
        pmat = (pos == jj.astype(F32)).astype(F32)
        sortv = lax.dot_general(vals, pmat, (((1,), (0,)), ((), ())),
                                preferred_element_type=F32,
                                precision=lax.Precision.HIGHEST)
        rid = jnp.clip(jnp.floor(sortv * (1.0 / TB)), 0.0, R - 1.0)
        eacc = jnp.zeros((1, _VS), F32)
        for mth in range(E):
            eacc = eacc + (sortv >= off_ex[:, mth:mth + 1]).astype(F32)
        eacc = eacc + (sortv >= float(N)).astype(F32)
        rid_ref[...] = rid.astype(jnp.int32)
        eid_ref[...] = jnp.clip(eacc - 1.0, 0.0, E - 1.0).astype(jnp.int32)
        offp = jnp.concatenate(
            [off_ex, jnp.full((1, 1), float(N), F32),
             jnp.zeros((1, 16 - E - 1), F32)], axis=1)
        off_ref[...] = offp.astype(jnp.int32)


def _ln_router(x2, g, b, rw, rb):
    return pl.pallas_call(
        _lnr_body,
        grid=(_NB,),
        in_specs=[
            pl.BlockSpec((TBA, D), lambda i: (i, 0)),
            pl.BlockSpec((1, D), lambda i: (0, 0)),
            pl.BlockSpec((1, D), lambda i: (0, 0)),
            pl.BlockSpec((D, E), lambda i: (0, 0)),
            pl.BlockSpec((1, E), lambda i: (0, 0)),
        ],
        out_specs=[
            pl.BlockSpec((TBA, 1), lambda i: (i, 0)),
            pl.BlockSpec((N, 1), lambda i: (0, 0)),
            pl.BlockSpec((1, _VS), lambda i: (0, 0)),
            pl.BlockSpec((1, _VS), lambda i: (0, 0)),
            pl.BlockSpec((1, 16), lambda i: (0, 0)),
        ],
        out_shape=[
            jax.ShapeDtypeStruct((N, 1), F32),
            jax.ShapeDtypeStruct((N, 1), jnp.int32),
            jax.ShapeDtypeStruct((1, _VS), jnp.int32),
            jax.ShapeDtypeStruct((1, _VS), jnp.int32),
            jax.ShapeDtypeStruct((1, 16), jnp.int32),
        ],
        scratch_shapes=[pltpu.VMEM((1, E), F32),
                        pltpu.VMEM((N, 1), jnp.int32),
                        pltpu.VMEM((N, 1), F32)],
    )(x2, g, b, rw, rb)


_NC, _NS = 2, 16
_NW = _NC * _NS
_BPW = N // _NW


def _sc_gather(xn, perm):
    mesh = plsc.VectorSubcoreMesh(core_axis_name="c", subcore_axis_name="s")

    @functools.partial(
        pl.kernel, mesh=mesh,
        out_type=jax.ShapeDtypeStruct((N, D), F32),
        scratch_types=[
            pltpu.VMEM((_BPW,), jnp.int32),
            pltpu.VMEM((_BPW, D), F32),
            pltpu.SemaphoreType.DMA,
        ],
    )
    def k(xn_hbm, perm_hbm, out_hbm, idx_v, rows_v, sem):
        wid = lax.axis_index("s") * _NC + lax.axis_index("c")
        base = wid * _BPW
        pltpu.sync_copy(perm_hbm.at[pl.ds(base, _BPW)], idx_v)
        pltpu.async_copy(xn_hbm.at[idx_v], rows_v, sem).wait()
        pltpu.sync_copy(rows_v, out_hbm.at[pl.ds(base, _BPW)])

    return k(xn, perm)


def _sc_scatter(ys, perm):
    mesh = plsc.VectorSubcoreMesh(core_axis_name="c", subcore_axis_name="s")

    @functools.partial(
        pl.kernel, mesh=mesh,
        out_type=jax.ShapeDtypeStruct((N, D), F32),
        scratch_types=[
            pltpu.VMEM((_BPW,), jnp.int32),
            pltpu.VMEM((_BPW, D), F32),
            pltpu.SemaphoreType.DMA,
        ],
    )
    def k(ys_hbm, perm_hbm, out_hbm, idx_v, rows_v, sem):
        wid = lax.axis_index("s") * _NC + lax.axis_index("c")
        base = wid * _BPW
        pltpu.sync_copy(perm_hbm.at[pl.ds(base, _BPW)], idx_v)
        pltpu.sync_copy(ys_hbm.at[pl.ds(base, _BPW)], rows_v)
        pltpu.async_copy(rows_v, out_hbm.at[idx_v], sem).wait()

    return k(ys, perm)


def _ffn_body(r_ref, e_ref, off_ref,
              xs_ref, g_ref, b_ref, w1_ref, b1_ref, w2_ref, b2_ref, out_ref):
    j = pl.program_id(0)
    r = r_ref[0, j]
    e = e_ref[0, j]
    x = xs_ref[...]
    mu = jnp.mean(x, axis=1, keepdims=True)
    var = jnp.mean((x - mu) ** 2, axis=1, keepdims=True)
    xn = (x - mu) / jnp.sqrt(var + 1e-5) * g_ref[...] + b_ref[...]
    h = jnp.dot(xn, w1_ref[0], preferred_element_type=F32) + b1_ref[pl.ds(e, 1)]
    h = jax.nn.gelu(h)
    y = jnp.dot(h, w2_ref[0], preferred_element_type=F32) + b2_ref[pl.ds(e, 1)]
    jp = jnp.maximum(j - 1, 0)
    dup = (j > 0) & (r == r_ref[0, jp]) & (e == e_ref[0, jp])
    lo = jnp.maximum(off_ref[0, e], r * TB)
    hi = jnp.where(dup, lo, jnp.minimum(off_ref[0, e + 1], (r + 1) * TB))
    rows = r * TB + lax.broadcasted_iota(jnp.int32, (TB, 1), 0)
    mask = (rows >= lo) & (rows < hi)
    contrib = jnp.where(mask, y, 0.0)
    first = jnp.logical_or(j == 0, r != r_ref[0, jp])

    @pl.when(first)
    def _():
        out_ref[...] = contrib

    @pl.when(jnp.logical_not(first))
    def _():
        out_ref[...] += contrib


def _ffn_grouped(r_ids, e_ids, off, xs, g, b, w1, b1, w2, b2):
    grid_spec = pltpu.PrefetchScalarGridSpec(
        num_scalar_prefetch=3,
        grid=(W,),
        in_specs=[
            pl.BlockSpec((TB, D), lambda j, r, e, off: (r[0, j], 0)),
            pl.BlockSpec((1, D), lambda j, r, e, off: (0, 0)),
            pl.BlockSpec((1, D), lambda j, r, e, off: (0, 0)),
            pl.BlockSpec((1, D, DFF), lambda j, r, e, off: (e[0, j], 0, 0)),
            pl.BlockSpec((E, DFF), lambda j, r, e, off: (0, 0)),
            pl.BlockSpec((1, DFF, D), lambda j, r, e, off: (e[0, j], 0, 0)),
            pl.BlockSpec((E, D), lambda j, r, e, off: (0, 0)),
        ],
        out_specs=pl.BlockSpec((TB, D), lambda j, r, e, off: (r[0, j], 0)),
    )
    return pl.pallas_call(
        _ffn_body,
        grid_spec=grid_spec,
        out_shape=jax.ShapeDtypeStruct((N, D), F32),
    )(r_ids, e_ids, off, xs, g, b, w1, b1, w2, b2)


_ID8 = (1.0, 0.0, 0.0, 0.0, 0.0, 0.0, 1.0, 0.0)


def _mm2x2(a, b):
    (a00r, a00i, a01r, a01i, a10r, a10i, a11r, a11i) = a
    (b00r, b00i, b01r, b01i, b10r, b10i, b11r, b11i) = b
    c00r = a00r * b00r - a00i * b00i + a01r * b10r - a01i * b10i
    c00i = a00r * b00i + a00i * b00r + a01r * b10i + a01i * b10r
    c01r = a00r * b01r - a00i * b01i + a01r * b11r - a01i * b11i
    c01i = a00r * b01i + a00i * b01r + a01r * b11i + a01i * b11r
    c10r = a10r * b00r - a10i * b00i + a11r * b10r - a11i * b10i
    c10i = a10r * b00i + a10i * b00r + a11r * b10i + a11i * b10r
    c11r = a10r * b01r - a10i * b01i + a11r * b11r - a11i * b11i
    c11i = a10r * b01i + a10i * b01r + a11r * b11i + a11i * b11r
    return (c00r, c00i, c01r, c01i, c10r, c10i, c11r, c11i)


def _mnorm(m):
    mx = m[0] * 0.0
    for t in m:
        mx = jnp.maximum(mx, jnp.abs(t))
    s = 1.0 / jnp.maximum(mx, 1e-30)
    return tuple(t * s for t in m)


def _shift(m, s, right):
    stacked = jnp.concatenate(m, axis=0)
    sh = pltpu.roll(stacked, s if right else C - s, 1)
    lanes = lax.broadcasted_iota(jnp.int32, (1, C), 1)
    cond = (lanes < s) if right else (lanes >= C - s)
    return tuple(jnp.where(cond, idv, sh[t:t + 1])
                 for t, idv in enumerate(_ID8))


def _chain(ar, forward):
    one = jnp.ones((1, C), F32)
    zero = jnp.zeros((1, C), F32)
    order = range(K) if forward else range(K - 1, -1, -1)
    out = [None] * K
    p = None
    for k in order:
        arr = ar[k:k + 1, :]
        if p is None:
            p = (arr, -one, -one, zero, one, zero, zero, zero)
        else:
            (p00r, p00i, p01r, p01i, p10r, p10i, p11r, p11i) = p
            n00r = arr * p00r + p00i - p10r
            n00i = arr * p00i - p00r - p10i
            n01r = arr * p01r + p01i - p11r
            n01i = arr * p01i - p01r - p11i
            p = (n00r, n00i, n01r, n01i, p00r, p00i, p01r, p01i)
        out[k] = p
    return out


def _prefix(m0, forward):
    x = _mnorm(m0)
    s = 1
    while s < C:
        xs = _shift(x, s, right=forward)
        x = _mnorm(_mm2x2(x, xs))
        s *= 2
    xs = _shift(x, 1, right=forward)
    return xs[0], xs[1], xs[4], xs[5]


def _bk_body(ys_ref, gate_ref, pw_ref, pb_ref, ow_ref, ob_ref, sc_ref, out_ref):
    ys = ys_ref[...]
    moe = ys * gate_ref[...]
    v = jnp.dot(moe, pw_ref[...], preferred_element_type=F32) + pb_ref[0, 0]
    hd = jnp.clip(v, -3.0, 3.0) - 2.0
    i2 = lax.broadcasted_iota(jnp.int32, (N, C), 0)
    c2 = lax.broadcasted_iota(jnp.int32, (N, C), 1)
    sel_c = jnp.where((i2 // K) == c2, 1.0, 0.0).astype(F32)
    k16 = lax.broadcasted_iota(jnp.int32, (K, N), 0)
    i16 = lax.broadcasted_iota(jnp.int32, (K, N), 1)
    w1sel = jnp.where((i16 % K) == k16, 1.0, 0.0).astype(F32)
    ar = jnp.dot(w1sel, sel_c * hd, preferred_element_type=F32)

    P = _chain(ar, forward=True)
    Q = _chain(ar, forward=False)
    unr, uni, udr, udi = _prefix(P[K - 1], forward=True)
    wnr, wni, wdr, wdi = _prefix(Q[0], forward=False)

    g_rows_r, g_rows_i = [], []
    for k in range(K):
        (p00r, p00i, p01r, p01i, p10r, p10i, p11r, p11i) = P[k]
        nLr = p00r * unr - p00i * uni + p01r * udr - p01i * udi
        nLi = p00r * uni + p00i * unr + p01r * udi + p01i * udr
        dLr = p10r * unr - p10i * uni + p11r * udr - p11i * udi
        dLi = p10r * uni + p10i * unr + p11r * udi + p11i * udr
        dd = jnp.maximum(dLr * dLr + dLi * dLi, 1e-30)
        Lr = (nLr * dLr + nLi * dLi) / dd
        Li = (nLi * dLr - nLr * dLi) / dd
        (q00r, q00i, q01r, q01i, q10r, q10i, q11r, q11i) = Q[k]
        nRr = q00r * wnr - q00i * wni + q01r * wdr - q01i * wdi
        nRi = q00r * wni + q00i * wnr + q01r * wdi + q01i * wdr
        dRr = q10r * wnr - q10i * wni + q11r * wdr - q11i * wdi
        dRi = q10r * wni + q10i * wnr + q11r * wdi + q11i * wdr
        ddr = jnp.maximum(dRr * dRr + dRi * dRi, 1e-30)
        Rr = (nRr * dRr + nRi * dRi) / ddr
        Ri = (nRi * dRr - nRr * dRi) / ddr
        sr = Lr + Rr - ar[k:k + 1, :]
        si = Li + Ri + 1.0
        den = jnp.maximum(sr * sr + si * si, 1e-30)
        g_rows_r.append(jnp.clip(sr / den, -10.0, 10.0))
        g_rows_i.append(jnp.clip(-si / den, -10.0, 10.0))
    re_g = jnp.concatenate(g_rows_r, axis=0)
    im_g = jnp.concatenate(g_rows_i, axis=0)

    w1t = jnp.where((lax.broadcasted_iota(jnp.int32, (N, K), 0) % K)
                    == lax.broadcasted_iota(jnp.int32, (N, K), 1),
                    1.0, 0.0).astype(F32)
    col_r = jnp.sum(jnp.dot(w1t, re_g, preferred_element_type=F32) * sel_c,
                    axis=1, keepdims=True)
    col_i = jnp.sum(jnp.dot(w1t, im_g, preferred_element_type=F32) * sel_c,
                    axis=1, keepdims=True)
    spec = col_r * ow_ref[0:1, :] + col_i * ow_ref[1:2, :] + ob_ref[...]
    out_ref[...] = (moe + sc_ref[0, 0] * spec)[None]


def _bk_final(yu, gate, pw, pb, ow, ob, bscale):
    return pl.pallas_call(
        _bk_body,
        grid=(1,),
        in_specs=[
            pl.BlockSpec((N, D), lambda i: (0, 0)),
            pl.BlockSpec((N, 1), lambda i: (0, 0)),
            pl.BlockSpec((D, 1), lambda i: (0, 0)),
            pl.BlockSpec((1, 1), lambda i: (0, 0)),
            pl.BlockSpec((2, D), lambda i: (0, 0)),
            pl.BlockSpec((1, D), lambda i: (0, 0)),
            pl.BlockSpec((1, 1), lambda i: (0, 0)),
        ],
        out_specs=pl.BlockSpec((1, N, D), lambda i: (0, 0, 0)),
        out_shape=jax.ShapeDtypeStruct((1, N, D), F32),
    )(yu, gate, pw, pb, ow, ob, bscale)


def kernel(x, ln_gamma, ln_beta, router_w, router_b, w1, b1, w2, b2,
           pproj_w, pproj_b, oproj_w, oproj_b, bk_scale):
    x2 = x.reshape(N, D)
    g2, b2r = ln_gamma.reshape(1, D), ln_beta.reshape(1, D)
    gate, s, r_ids, e_ids, off = _ln_router(x2, g2, b2r, router_w,
                                            router_b.reshape(1, E))
    sf = s.reshape(N)
    xs = _sc_scatter(x2, sf)
    ys = _ffn_grouped(r_ids, e_ids, off, xs, g2, b2r, w1, b1, w2, b2)
    yu = _sc_gather(ys, sf)
    return _bk_final(yu, gate, pproj_w, pproj_b.reshape(1, 1), oproj_w,
                     oproj_b.reshape(1, D), jnp.asarray(bk_scale).reshape(1, 1))


# speedup vs baseline: 1.1348x; 1.0450x over previous
import functools

import jax
import jax.numpy as jnp
from jax import lax
from jax.experimental import pallas as pl
from jax.experimental.pallas import tpu as pltpu
from jax.experimental.pallas import tpu_sc as plsc

N, D, E, DFF = 2048, 768, 8, 3072
TB = 256
R = N // TB
W = R + E - 1
TBA = 512
K = 16
C = N // K
F32 = jnp.float32


_NB = N // TBA
_VS = 32


def _lnr_body(x_ref, g_ref, b_ref, rw_ref, rb_ref,
              gate_ref, s_ref, rid_ref, eid_ref, off_ref,
              acc_ref, idx_scr, rank_scr):
    i = pl.program_id(0)
    x = x_ref[...]
    mu = jnp.mean(x, axis=1, keepdims=True)
    var = jnp.mean((x - mu) ** 2, axis=1, keepdims=True)
    xn = (x - mu) / jnp.sqrt(var + 1e-5) * g_ref[...] + b_ref[...]
    logits = jnp.dot(xn, rw_ref[...], preferred_element_type=F32) + rb_ref[...]
    m = jnp.max(logits, axis=1, keepdims=True)
    gate_ref[...] = 1.0 / jnp.sum(jnp.exp(logits - m), axis=1, keepdims=True)
    e_iota = lax.broadcasted_iota(jnp.int32, (TBA, E), 1)
    idxv = jnp.min(jnp.where(logits >= m, e_iota, E), axis=1, keepdims=True)
    onehot = (idxv == e_iota).astype(F32)
    r1 = lax.broadcasted_iota(jnp.int32, (TBA, TBA), 0)
    c1 = lax.broadcasted_iota(jnp.int32, (TBA, TBA), 1)
    tri = (c1 <= r1).astype(F32)
    cum = jnp.dot(tri, onehot, preferred_element_type=F32)

    @pl.when(i == 0)
    def _():
        acc_ref[...] = jnp.zeros((1, E), F32)

    base = acc_ref[...]
    rank = jnp.sum(onehot * (cum + base), axis=1, keepdims=True) - 1.0
    idx_scr[pl.ds(i * TBA, TBA), :] = idxv.astype(jnp.int32)
    rank_scr[pl.ds(i * TBA, TBA), :] = rank
    acc_ref[...] = base + cum[TBA - 1:TBA, :]

    @pl.when(i == _NB - 1)
    def _():
        counts = acc_ref[...]
        ku = lax.broadcasted_iota(jnp.int32, (E, E), 0)
        eu = lax.broadcasted_iota(jnp.int32, (E, E), 1)
        upper = (ku < eu).astype(F32)
        off_ex = jnp.dot(counts, upper, preferred_element_type=F32,
                         precision=lax.Precision.HIGHEST)
        idx_all = idx_scr[...]
        oh_all = (idx_all == lax.broadcasted_iota(jnp.int32, (N, E), 1))
        s = rank_scr[...] + jnp.sum(oh_all.astype(F32) * off_ex, axis=1,
                                    keepdims=True)
        s_ref[...] = s.astype(jnp.int32)
        blocks = (lax.broadcasted_iota(jnp.int32, (1, R), 1) * TB).astype(F32)
        huge = jnp.full((1, _VS - (E - 1) - R), 1e9, F32)
        vals = jnp.concatenate([off_ex[:, 1:E], blocks, huge], axis=1)
        ident = (lax.broadcasted_iota(jnp.int32, (_VS, _VS), 0)
                 == lax.broadcasted_iota(jnp.int32, (_VS, _VS), 1)).astype(F32)
        vals_col = lax.dot_general(ident, vals, (((1,), (1,)), ((), ())),
                                   preferred_element_type=F32,
                                   precision=lax.Precision.HIGHEST)
        ii = lax.broadcasted_iota(jnp.int32, (_VS, _VS), 0)
        jj = lax.broadcasted_iota(jnp.int32, (_VS, _VS), 1)
        less = (vals < vals_col) | ((vals == vals_col) & (jj < ii))
        pos = jnp.sum(less.astype(F32), axis=1, keepdims=True)
        pmat = (pos == jj.astype(F32)).astype(F32)
        sortv = lax.dot_general(vals, pmat, (((1,), (0,)), ((), ())),
                                preferred_element_type=F32,
                                precision=lax.Precision.HIGHEST)
        rid = jnp.clip(jnp.floor(sortv * (1.0 / TB)), 0.0, R - 1.0)
        eacc = jnp.zeros((1, _VS), F32)
        for mth in range(E):
            eacc = eacc + (sortv >= off_ex[:, mth:mth + 1]).astype(F32)
        eacc = eacc + (sortv >= float(N)).astype(F32)
        rid_ref[...] = rid.astype(jnp.int32)
        eid_ref[...] = jnp.clip(eacc - 1.0, 0.0, E - 1.0).astype(jnp.int32)
        offp = jnp.concatenate(
            [off_ex, jnp.full((1, 1), float(N), F32),
             jnp.zeros((1, 16 - E - 1), F32)], axis=1)
        off_ref[...] = offp.astype(jnp.int32)


def _ln_router(x2, g, b, rw, rb):
    return pl.pallas_call(
        _lnr_body,
        grid=(_NB,),
        in_specs=[
            pl.BlockSpec((TBA, D), lambda i: (i, 0)),
            pl.BlockSpec((1, D), lambda i: (0, 0)),
            pl.BlockSpec((1, D), lambda i: (0, 0)),
            pl.BlockSpec((D, E), lambda i: (0, 0)),
            pl.BlockSpec((1, E), lambda i: (0, 0)),
        ],
        out_specs=[
            pl.BlockSpec((TBA, 1), lambda i: (i, 0)),
            pl.BlockSpec((N, 1), lambda i: (0, 0)),
            pl.BlockSpec((1, _VS), lambda i: (0, 0)),
            pl.BlockSpec((1, _VS), lambda i: (0, 0)),
            pl.BlockSpec((1, 16), lambda i: (0, 0)),
        ],
        out_shape=[
            jax.ShapeDtypeStruct((N, 1), F32),
            jax.ShapeDtypeStruct((N, 1), jnp.int32),
            jax.ShapeDtypeStruct((1, _VS), jnp.int32),
            jax.ShapeDtypeStruct((1, _VS), jnp.int32),
            jax.ShapeDtypeStruct((1, 16), jnp.int32),
        ],
        scratch_shapes=[pltpu.VMEM((1, E), F32),
                        pltpu.VMEM((N, 1), jnp.int32),
                        pltpu.VMEM((N, 1), F32)],
    )(x2, g, b, rw, rb)


_NC, _NS = 2, 16
_NW = _NC * _NS
_BPW = N // _NW


def _sc_gather(xn, perm):
    mesh = plsc.VectorSubcoreMesh(core_axis_name="c", subcore_axis_name="s")

    @functools.partial(
        pl.kernel, mesh=mesh,
        out_type=jax.ShapeDtypeStruct((N, D), F32),
        scratch_types=[
            pltpu.VMEM((_BPW,), jnp.int32),
            pltpu.VMEM((_BPW, D), F32),
            pltpu.SemaphoreType.DMA,
        ],
    )
    def k(xn_hbm, perm_hbm, out_hbm, idx_v, rows_v, sem):
        wid = lax.axis_index("s") * _NC + lax.axis_index("c")
        base = wid * _BPW
        pltpu.sync_copy(perm_hbm.at[pl.ds(base, _BPW)], idx_v)
        pltpu.async_copy(xn_hbm.at[idx_v], rows_v, sem).wait()
        pltpu.sync_copy(rows_v, out_hbm.at[pl.ds(base, _BPW)])

    return k(xn, perm)


def _sc_scatter(ys, perm):
    mesh = plsc.VectorSubcoreMesh(core_axis_name="c", subcore_axis_name="s")

    @functools.partial(
        pl.kernel, mesh=mesh,
        out_type=jax.ShapeDtypeStruct((N, D), F32),
        scratch_types=[
            pltpu.VMEM((_BPW,), jnp.int32),
            pltpu.VMEM((_BPW, D), F32),
            pltpu.SemaphoreType.DMA,
        ],
    )
    def k(ys_hbm, perm_hbm, out_hbm, idx_v, rows_v, sem):
        wid = lax.axis_index("s") * _NC + lax.axis_index("c")
        base = wid * _BPW
        pltpu.sync_copy(perm_hbm.at[pl.ds(base, _BPW)], idx_v)
        pltpu.sync_copy(ys_hbm.at[pl.ds(base, _BPW)], rows_v)
        pltpu.async_copy(rows_v, out_hbm.at[idx_v], sem).wait()

    return k(ys, perm)


def _ffn_body(r_ref, e_ref, off_ref,
              xs_ref, g_ref, b_ref, w1_ref, b1_ref, w2_ref, b2_ref, out_ref):
    j = pl.program_id(0)
    r = r_ref[0, j]
    e = e_ref[0, j]
    x = xs_ref[...]
    mu = jnp.mean(x, axis=1, keepdims=True)
    var = jnp.mean((x - mu) ** 2, axis=1, keepdims=True)
    xn = (x - mu) / jnp.sqrt(var + 1e-5) * g_ref[...] + b_ref[...]
    h = jnp.dot(xn, w1_ref[0], preferred_element_type=F32) + b1_ref[pl.ds(e, 1)]
    h = jax.nn.gelu(h)
    y = jnp.dot(h, w2_ref[0], preferred_element_type=F32) + b2_ref[pl.ds(e, 1)]
    jp = jnp.maximum(j - 1, 0)
    dup = (j > 0) & (r == r_ref[0, jp]) & (e == e_ref[0, jp])
    lo = jnp.maximum(off_ref[0, e], r * TB)
    hi = jnp.where(dup, lo, jnp.minimum(off_ref[0, e + 1], (r + 1) * TB))
    rows = r * TB + lax.broadcasted_iota(jnp.int32, (TB, 1), 0)
    mask = (rows >= lo) & (rows < hi)
    contrib = jnp.where(mask, y, 0.0)
    first = jnp.logical_or(j == 0, r != r_ref[0, jp])

    @pl.when(first)
    def _():
        out_ref[...] = contrib

    @pl.when(jnp.logical_not(first))
    def _():
        out_ref[...] += contrib


def _ffn_grouped(r_ids, e_ids, off, xs, g, b, w1, b1, w2, b2):
    grid_spec = pltpu.PrefetchScalarGridSpec(
        num_scalar_prefetch=3,
        grid=(W,),
        in_specs=[
            pl.BlockSpec((TB, D), lambda j, r, e, off: (r[0, j], 0)),
            pl.BlockSpec((1, D), lambda j, r, e, off: (0, 0)),
            pl.BlockSpec((1, D), lambda j, r, e, off: (0, 0)),
            pl.BlockSpec((1, D, DFF), lambda j, r, e, off: (e[0, j], 0, 0)),
            pl.BlockSpec((E, DFF), lambda j, r, e, off: (0, 0)),
            pl.BlockSpec((1, DFF, D), lambda j, r, e, off: (e[0, j], 0, 0)),
            pl.BlockSpec((E, D), lambda j, r, e, off: (0, 0)),
        ],
        out_specs=pl.BlockSpec((TB, D), lambda j, r, e, off: (r[0, j], 0)),
    )
    return pl.pallas_call(
        _ffn_body,
        grid_spec=grid_spec,
        out_shape=jax.ShapeDtypeStruct((N, D), F32),
    )(r_ids, e_ids, off, xs, g, b, w1, b1, w2, b2)


_ID8 = (1.0, 0.0, 0.0, 0.0, 0.0, 0.0, 1.0, 0.0)


def _mm2x2(a, b):
    (a00r, a00i, a01r, a01i, a10r, a10i, a11r, a11i) = a
    (b00r, b00i, b01r, b01i, b10r, b10i, b11r, b11i) = b
    c00r = a00r * b00r - a00i * b00i + a01r * b10r - a01i * b10i
    c00i = a00r * b00i + a00i * b00r + a01r * b10i + a01i * b10r
    c01r = a00r * b01r - a00i * b01i + a01r * b11r - a01i * b11i
    c01i = a00r * b01i + a00i * b01r + a01r * b11i + a01i * b11r
    c10r = a10r * b00r - a10i * b00i + a11r * b10r - a11i * b10i
    c10i = a10r * b00i + a10i * b00r + a11r * b10i + a11i * b10r
    c11r = a10r * b01r - a10i * b01i + a11r * b11r - a11i * b11i
    c11i = a10r * b01i + a10i * b01r + a11r * b11i + a11i * b11r
    return (c00r, c00i, c01r, c01i, c10r, c10i, c11r, c11i)


def _mnorm(m):
    mx = m[0] * 0.0
    for t in m:
        mx = jnp.maximum(mx, jnp.abs(t))
    s = 1.0 / jnp.maximum(mx, 1e-30)
    return tuple(t * s for t in m)


def _shift(m, s, right):
    stacked = jnp.concatenate(m, axis=0)
    sh = pltpu.roll(stacked, s if right else C - s, 1)
    lanes = lax.broadcasted_iota(jnp.int32, (1, C), 1)
    cond = (lanes < s) if right else (lanes >= C - s)
    return tuple(jnp.where(cond, idv, sh[t:t + 1])
                 for t, idv in enumerate(_ID8))


def _chain(ar, forward):
    one = jnp.ones((1, C), F32)
    zero = jnp.zeros((1, C), F32)
    order = range(K) if forward else range(K - 1, -1, -1)
    out = [None] * K
    p = None
    for k in order:
        arr = ar[k:k + 1, :]
        if p is None:
            p = (arr, -one, -one, zero, one, zero, zero, zero)
        else:
            (p00r, p00i, p01r, p01i, p10r, p10i, p11r, p11i) = p
            n00r = arr * p00r + p00i - p10r
            n00i = arr * p00i - p00r - p10i
            n01r = arr * p01r + p01i - p11r
            n01i = arr * p01i - p01r - p11i
            p = (n00r, n00i, n01r, n01i, p00r, p00i, p01r, p01i)
        out[k] = p
    return out


def _prefix(m0, forward):
    x = _mnorm(m0)
    s = 1
    while s < C:
        xs = _shift(x, s, right=forward)
        x = _mnorm(_mm2x2(x, xs))
        s *= 2
    xs = _shift(x, 1, right=forward)
    return xs[0], xs[1], xs[4], xs[5]


def _bk_body(ys_ref, gate_ref, pw_ref, pb_ref, ow_ref, ob_ref, sc_ref, out_ref):
    ys = ys_ref[...]
    moe = ys * gate_ref[...]
    v = jnp.dot(moe, pw_ref[...], preferred_element_type=F32) + pb_ref[0, 0]
    hd = jnp.clip(v, -3.0, 3.0) - 2.0
    i2 = lax.broadcasted_iota(jnp.int32, (N, C), 0)
    c2 = lax.broadcasted_iota(jnp.int32, (N, C), 1)
    sel_c = jnp.where((i2 // K) == c2, 1.0, 0.0).astype(F32)
    k16 = lax.broadcasted_iota(jnp.int32, (K, N), 0)
    i16 = lax.broadcasted_iota(jnp.int32, (K, N), 1)
    w1sel = jnp.where((i16 % K) == k16, 1.0, 0.0).astype(F32)
    ar = jnp.dot(w1sel, sel_c * hd, preferred_element_type=F32)

    P = _chain(ar, forward=True)
    Q = _chain(ar, forward=False)
    unr, uni, udr, udi = _prefix(P[K - 1], forward=True)
    wnr, wni, wdr, wdi = _prefix(Q[0], forward=False)

    g_rows_r, g_rows_i = [], []
    for k in range(K):
        (p00r, p00i, p01r, p01i, p10r, p10i, p11r, p11i) = P[k]
        nLr = p00r * unr - p00i * uni + p01r * udr - p01i * udi
        nLi = p00r * uni + p00i * unr + p01r * udi + p01i * udr
        dLr = p10r * unr - p10i * uni + p11r * udr - p11i * udi
        dLi = p10r * uni + p10i * unr + p11r * udi + p11i * udr
        dd = jnp.maximum(dLr * dLr + dLi * dLi, 1e-30)
        Lr = (nLr * dLr + nLi * dLi) / dd
        Li = (nLi * dLr - nLr * dLi) / dd
        (q00r, q00i, q01r, q01i, q10r, q10i, q11r, q11i) = Q[k]
        nRr = q00r * wnr - q00i * wni + q01r * wdr - q01i * wdi
        nRi = q00r * wni + q00i * wnr + q01r * wdi + q01i * wdr
        dRr = q10r * wnr - q10i * wni + q11r * wdr - q11i * wdi
        dRi = q10r * wni + q10i * wnr + q11r * wdi + q11i * wdr
        ddr = jnp.maximum(dRr * dRr + dRi * dRi, 1e-30)
        Rr = (nRr * dRr + nRi * dRi) / ddr
        Ri = (nRi * dRr - nRr * dRi) / ddr
        sr = Lr + Rr - ar[k:k + 1, :]
        si = Li + Ri + 1.0
        den = jnp.maximum(sr * sr + si * si, 1e-30)
        g_rows_r.append(jnp.clip(sr / den, -10.0, 10.0))
        g_rows_i.append(jnp.clip(-si / den, -10.0, 10.0))
    re_g = jnp.concatenate(g_rows_r, axis=0)
    im_g = jnp.concatenate(g_rows_i, axis=0)

    w1t = jnp.where((lax.broadcasted_iota(jnp.int32, (N, K), 0) % K)
                    == lax.broadcasted_iota(jnp.int32, (N, K), 1),
                    1.0, 0.0).astype(F32)
    col_r = jnp.sum(jnp.dot(w1t, re_g, preferred_element_type=F32) * sel_c,
                    axis=1, keepdims=True)
    col_i = jnp.sum(jnp.dot(w1t, im_g, preferred_element_type=F32) * sel_c,
                    axis=1, keepdims=True)
    spec = col_r * ow_ref[0:1, :] + col_i * ow_ref[1:2, :] + ob_ref[...]
    out_ref[...] = (moe + sc_ref[0, 0] * spec)[None]


def _bk_final(yu, gate, pw, pb, ow, ob, bscale):
    return pl.pallas_call(
        _bk_body,
        grid=(1,),
        in_specs=[
            pl.BlockSpec((N, D), lambda i: (0, 0)),
            pl.BlockSpec((N, 1), lambda i: (0, 0)),
            pl.BlockSpec((D, 1), lambda i: (0, 0)),
            pl.BlockSpec((1, 1), lambda i: (0, 0)),
            pl.BlockSpec((2, D), lambda i: (0, 0)),
            pl.BlockSpec((1, D), lambda i: (0, 0)),
            pl.BlockSpec((1, 1), lambda i: (0, 0)),
        ],
        out_specs=pl.BlockSpec((1, N, D), lambda i: (0, 0, 0)),
        out_shape=jax.ShapeDtypeStruct((1, N, D), F32),
    )(yu, gate, pw, pb, ow, ob, bscale)


def kernel(x, ln_gamma, ln_beta, router_w, router_b, w1, b1, w2, b2,
           pproj_w, pproj_b, oproj_w, oproj_b, bk_scale):
    x2 = x.reshape(N, D)
    g2, b2r = ln_gamma.reshape(1, D), ln_beta.reshape(1, D)
    gate, s, r_ids, e_ids, off = _ln_router(x2, g2, b2r, router_w,
                                            router_b.reshape(1, E))
    sf = s.reshape(N)
    xs = _sc_scatter(x2, sf)
    ys = _ffn_grouped(r_ids, e_ids, off, xs, g2, b2r, w1, b1, w2, b2)
    yu = _sc_gather(ys, sf)
    return _bk_final(yu, gate, pproj_w, pproj_b.reshape(1, 1), oproj_w,
                     oproj_b.reshape(1, D), jnp.asarray(bk_scale).reshape(1, 1))


# speedup vs baseline: 1.1660x; 1.0275x over previous
import functools

import jax
import jax.numpy as jnp
from jax import lax
from jax.experimental import pallas as pl
from jax.experimental.pallas import tpu as pltpu
from jax.experimental.pallas import tpu_sc as plsc

N, D, E, DFF = 2048, 768, 8, 3072
TB = 512
R = N // TB
W = R + E - 1
TBA = 512
K = 16
C = N // K
F32 = jnp.float32


_NB = N // TBA
_VS = 32


def _lnr_body(x_ref, g_ref, b_ref, rw_ref, rb_ref,
              gate_ref, s_ref, rid_ref, eid_ref, off_ref,
              acc_ref, idx_scr, rank_scr):
    i = pl.program_id(0)
    x = x_ref[...]
    mu = jnp.mean(x, axis=1, keepdims=True)
    var = jnp.mean((x - mu) ** 2, axis=1, keepdims=True)
    xn = (x - mu) / jnp.sqrt(var + 1e-5) * g_ref[...] + b_ref[...]
    logits = jnp.dot(xn, rw_ref[...], preferred_element_type=F32) + rb_ref[...]
    m = jnp.max(logits, axis=1, keepdims=True)
    gate_ref[...] = 1.0 / jnp.sum(jnp.exp(logits - m), axis=1, keepdims=True)
    e_iota = lax.broadcasted_iota(jnp.int32, (TBA, E), 1)
    idxv = jnp.min(jnp.where(logits >= m, e_iota, E), axis=1, keepdims=True)
    onehot = (idxv == e_iota).astype(F32)
    r1 = lax.broadcasted_iota(jnp.int32, (TBA, TBA), 0)
    c1 = lax.broadcasted_iota(jnp.int32, (TBA, TBA), 1)
    tri = (c1 <= r1).astype(F32)
    cum = jnp.dot(tri, onehot, preferred_element_type=F32)

    @pl.when(i == 0)
    def _():
        acc_ref[...] = jnp.zeros((1, E), F32)

    base = acc_ref[...]
    rank = jnp.sum(onehot * (cum + base), axis=1, keepdims=True) - 1.0
    idx_scr[pl.ds(i * TBA, TBA), :] = idxv.astype(jnp.int32)
    rank_scr[pl.ds(i * TBA, TBA), :] = rank
    acc_ref[...] = base + cum[TBA - 1:TBA, :]

    @pl.when(i == _NB - 1)
    def _():
        counts = acc_ref[...]
        ku = lax.broadcasted_iota(jnp.int32, (E, E), 0)
        eu = lax.broadcasted_iota(jnp.int32, (E, E), 1)
        upper = (ku < eu).astype(F32)
        off_ex = jnp.dot(counts, upper, preferred_element_type=F32,
                         precision=lax.Precision.HIGHEST)
        idx_all = idx_scr[...]
        oh_all = (idx_all == lax.broadcasted_iota(jnp.int32, (N, E), 1))
        s = rank_scr[...] + jnp.sum(oh_all.astype(F32) * off_ex, axis=1,
                                    keepdims=True)
        s_ref[...] = s.astype(jnp.int32)
        blocks = (lax.broadcasted_iota(jnp.int32, (1, R), 1) * TB).astype(F32)
        huge = jnp.full((1, _VS - (E - 1) - R), 1e9, F32)
        vals = jnp.concatenate([off_ex[:, 1:E], blocks, huge], axis=1)
        ident = (lax.broadcasted_iota(jnp.int32, (_VS, _VS), 0)
                 == lax.broadcasted_iota(jnp.int32, (_VS, _VS), 1)).astype(F32)
        vals_col = lax.dot_general(ident, vals, (((1,), (1,)), ((), ())),
                                   preferred_element_type=F32,
                                   precision=lax.Precision.HIGHEST)
        ii = lax.broadcasted_iota(jnp.int32, (_VS, _VS), 0)
        jj = lax.broadcasted_iota(jnp.int32, (_VS, _VS), 1)
        less = (vals < vals_col) | ((vals == vals_col) & (jj < ii))
        pos = jnp.sum(less.astype(F32), axis=1, keepdims=True)
        pmat = (pos == jj.astype(F32)).astype(F32)
        sortv = lax.dot_general(vals, pmat, (((1,), (0,)), ((), ())),
                                preferred_element_type=F32,
                                precision=lax.Precision.HIGHEST)
        rid = jnp.clip(jnp.floor(sortv * (1.0 / TB)), 0.0, R - 1.0)
        eacc = jnp.zeros((1, _VS), F32)
        for mth in range(E):
            eacc = eacc + (sortv >= off_ex[:, mth:mth + 1]).astype(F32)
        eacc = eacc + (sortv >= float(N)).astype(F32)
        rid_ref[...] = rid.astype(jnp.int32)
        eid_ref[...] = jnp.clip(eacc - 1.0, 0.0, E - 1.0).astype(jnp.int32)
        offp = jnp.concatenate(
            [off_ex, jnp.full((1, 1), float(N), F32),
             jnp.zeros((1, 16 - E - 1), F32)], axis=1)
        off_ref[...] = offp.astype(jnp.int32)


def _ln_router(x2, g, b, rw, rb):
    return pl.pallas_call(
        _lnr_body,
        grid=(_NB,),
        in_specs=[
            pl.BlockSpec((TBA, D), lambda i: (i, 0)),
            pl.BlockSpec((1, D), lambda i: (0, 0)),
            pl.BlockSpec((1, D), lambda i: (0, 0)),
            pl.BlockSpec((D, E), lambda i: (0, 0)),
            pl.BlockSpec((1, E), lambda i: (0, 0)),
        ],
        out_specs=[
            pl.BlockSpec((TBA, 1), lambda i: (i, 0)),
            pl.BlockSpec((N, 1), lambda i: (0, 0)),
            pl.BlockSpec((1, _VS), lambda i: (0, 0)),
            pl.BlockSpec((1, _VS), lambda i: (0, 0)),
            pl.BlockSpec((1, 16), lambda i: (0, 0)),
        ],
        out_shape=[
            jax.ShapeDtypeStruct((N, 1), F32),
            jax.ShapeDtypeStruct((N, 1), jnp.int32),
            jax.ShapeDtypeStruct((1, _VS), jnp.int32),
            jax.ShapeDtypeStruct((1, _VS), jnp.int32),
            jax.ShapeDtypeStruct((1, 16), jnp.int32),
        ],
        scratch_shapes=[pltpu.VMEM((1, E), F32),
                        pltpu.VMEM((N, 1), jnp.int32),
                        pltpu.VMEM((N, 1), F32)],
    )(x2, g, b, rw, rb)


_NC, _NS = 2, 16
_NW = _NC * _NS
_BPW = N // _NW


def _sc_gather(xn, perm):
    mesh = plsc.VectorSubcoreMesh(core_axis_name="c", subcore_axis_name="s")

    @functools.partial(
        pl.kernel, mesh=mesh,
        out_type=jax.ShapeDtypeStruct((N, D), F32),
        scratch_types=[
            pltpu.VMEM((_BPW,), jnp.int32),
            pltpu.VMEM((_BPW, D), F32),
            pltpu.SemaphoreType.DMA,
        ],
    )
    def k(xn_hbm, perm_hbm, out_hbm, idx_v, rows_v, sem):
        wid = lax.axis_index("s") * _NC + lax.axis_index("c")
        base = wid * _BPW
        pltpu.sync_copy(perm_hbm.at[pl.ds(base, _BPW)], idx_v)
        pltpu.async_copy(xn_hbm.at[idx_v], rows_v, sem).wait()
        pltpu.sync_copy(rows_v, out_hbm.at[pl.ds(base, _BPW)])

    return k(xn, perm)


def _sc_scatter(ys, perm):
    mesh = plsc.VectorSubcoreMesh(core_axis_name="c", subcore_axis_name="s")

    @functools.partial(
        pl.kernel, mesh=mesh,
        out_type=jax.ShapeDtypeStruct((N, D), F32),
        scratch_types=[
            pltpu.VMEM((_BPW,), jnp.int32),
            pltpu.VMEM((_BPW, D), F32),
            pltpu.SemaphoreType.DMA,
        ],
    )
    def k(ys_hbm, perm_hbm, out_hbm, idx_v, rows_v, sem):
        wid = lax.axis_index("s") * _NC + lax.axis_index("c")
        base = wid * _BPW
        pltpu.sync_copy(perm_hbm.at[pl.ds(base, _BPW)], idx_v)
        pltpu.sync_copy(ys_hbm.at[pl.ds(base, _BPW)], rows_v)
        pltpu.async_copy(rows_v, out_hbm.at[idx_v], sem).wait()

    return k(ys, perm)


def _ffn_body(r_ref, e_ref, off_ref,
              xs_ref, g_ref, b_ref, w1_ref, b1_ref, w2_ref, b2_ref, out_ref):
    j = pl.program_id(0)
    r = r_ref[0, j]
    e = e_ref[0, j]
    x = xs_ref[...]
    mu = jnp.mean(x, axis=1, keepdims=True)
    var = jnp.mean((x - mu) ** 2, axis=1, keepdims=True)
    xn = (x - mu) / jnp.sqrt(var + 1e-5) * g_ref[...] + b_ref[...]
    h = jnp.dot(xn, w1_ref[0], preferred_element_type=F32) + b1_ref[pl.ds(e, 1)]
    h = jax.nn.gelu(h)
    y = jnp.dot(h, w2_ref[0], preferred_element_type=F32) + b2_ref[pl.ds(e, 1)]
    jp = jnp.maximum(j - 1, 0)
    dup = (j > 0) & (r == r_ref[0, jp]) & (e == e_ref[0, jp])
    lo = jnp.maximum(off_ref[0, e], r * TB)
    hi = jnp.where(dup, lo, jnp.minimum(off_ref[0, e + 1], (r + 1) * TB))
    rows = r * TB + lax.broadcasted_iota(jnp.int32, (TB, 1), 0)
    mask = (rows >= lo) & (rows < hi)
    contrib = jnp.where(mask, y, 0.0)
    first = jnp.logical_or(j == 0, r != r_ref[0, jp])

    @pl.when(first)
    def _():
        out_ref[...] = contrib

    @pl.when(jnp.logical_not(first))
    def _():
        out_ref[...] += contrib


def _ffn_grouped(r_ids, e_ids, off, xs, g, b, w1, b1, w2, b2):
    grid_spec = pltpu.PrefetchScalarGridSpec(
        num_scalar_prefetch=3,
        grid=(W,),
        in_specs=[
            pl.BlockSpec((TB, D), lambda j, r, e, off: (r[0, j], 0)),
            pl.BlockSpec((1, D), lambda j, r, e, off: (0, 0)),
            pl.BlockSpec((1, D), lambda j, r, e, off: (0, 0)),
            pl.BlockSpec((1, D, DFF), lambda j, r, e, off: (e[0, j], 0, 0)),
            pl.BlockSpec((E, DFF), lambda j, r, e, off: (0, 0)),
            pl.BlockSpec((1, DFF, D), lambda j, r, e, off: (e[0, j], 0, 0)),
            pl.BlockSpec((E, D), lambda j, r, e, off: (0, 0)),
        ],
        out_specs=pl.BlockSpec((TB, D), lambda j, r, e, off: (r[0, j], 0)),
    )
    return pl.pallas_call(
        _ffn_body,
        grid_spec=grid_spec,
        out_shape=jax.ShapeDtypeStruct((N, D), F32),
    )(r_ids, e_ids, off, xs, g, b, w1, b1, w2, b2)


_ID8 = (1.0, 0.0, 0.0, 0.0, 0.0, 0.0, 1.0, 0.0)


def _mm2x2(a, b):
    (a00r, a00i, a01r, a01i, a10r, a10i, a11r, a11i) = a
    (b00r, b00i, b01r, b01i, b10r, b10i, b11r, b11i) = b
    c00r = a00r * b00r - a00i * b00i + a01r * b10r - a01i * b10i
    c00i = a00r * b00i + a00i * b00r + a01r * b10i + a01i * b10r
    c01r = a00r * b01r - a00i * b01i + a01r * b11r - a01i * b11i
    c01i = a00r * b01i + a00i * b01r + a01r * b11i + a01i * b11r
    c10r = a10r * b00r - a10i * b00i + a11r * b10r - a11i * b10i
    c10i = a10r * b00i + a10i * b00r + a11r * b10i + a11i * b10r
    c11r = a10r * b01r - a10i * b01i + a11r * b11r - a11i * b11i
    c11i = a10r * b01i + a10i * b01r + a11r * b11i + a11i * b11r
    return (c00r, c00i, c01r, c01i, c10r, c10i, c11r, c11i)


def _mnorm(m):
    mx = m[0] * 0.0
    for t in m:
        mx = jnp.maximum(mx, jnp.abs(t))
    s = 1.0 / jnp.maximum(mx, 1e-30)
    return tuple(t * s for t in m)


def _shift(m, s, right):
    stacked = jnp.concatenate(m, axis=0)
    sh = pltpu.roll(stacked, s if right else C - s, 1)
    lanes = lax.broadcasted_iota(jnp.int32, (1, C), 1)
    cond = (lanes < s) if right else (lanes >= C - s)
    return tuple(jnp.where(cond, idv, sh[t:t + 1])
                 for t, idv in enumerate(_ID8))


def _chain(ar, forward):
    one = jnp.ones((1, C), F32)
    zero = jnp.zeros((1, C), F32)
    order = range(K) if forward else range(K - 1, -1, -1)
    out = [None] * K
    p = None
    for k in order:
        arr = ar[k:k + 1, :]
        if p is None:
            p = (arr, -one, -one, zero, one, zero, zero, zero)
        else:
            (p00r, p00i, p01r, p01i, p10r, p10i, p11r, p11i) = p
            n00r = arr * p00r + p00i - p10r
            n00i = arr * p00i - p00r - p10i
            n01r = arr * p01r + p01i - p11r
            n01i = arr * p01i - p01r - p11i
            p = (n00r, n00i, n01r, n01i, p00r, p00i, p01r, p01i)
        out[k] = p
    return out


def _prefix(m0, forward):
    x = _mnorm(m0)
    s = 1
    while s < C:
        xs = _shift(x, s, right=forward)
        x = _mnorm(_mm2x2(x, xs))
        s *= 2
    xs = _shift(x, 1, right=forward)
    return xs[0], xs[1], xs[4], xs[5]


def _bk_body(ys_ref, gate_ref, pw_ref, pb_ref, ow_ref, ob_ref, sc_ref, out_ref):
    ys = ys_ref[...]
    moe = ys * gate_ref[...]
    v = jnp.dot(moe, pw_ref[...], preferred_element_type=F32) + pb_ref[0, 0]
    hd = jnp.clip(v, -3.0, 3.0) - 2.0
    i2 = lax.broadcasted_iota(jnp.int32, (N, C), 0)
    c2 = lax.broadcasted_iota(jnp.int32, (N, C), 1)
    sel_c = jnp.where((i2 // K) == c2, 1.0, 0.0).astype(F32)
    k16 = lax.broadcasted_iota(jnp.int32, (K, N), 0)
    i16 = lax.broadcasted_iota(jnp.int32, (K, N), 1)
    w1sel = jnp.where((i16 % K) == k16, 1.0, 0.0).astype(F32)
    ar = jnp.dot(w1sel, sel_c * hd, preferred_element_type=F32)

    P = _chain(ar, forward=True)
    Q = _chain(ar, forward=False)
    unr, uni, udr, udi = _prefix(P[K - 1], forward=True)
    wnr, wni, wdr, wdi = _prefix(Q[0], forward=False)

    g_rows_r, g_rows_i = [], []
    for k in range(K):
        (p00r, p00i, p01r, p01i, p10r, p10i, p11r, p11i) = P[k]
        nLr = p00r * unr - p00i * uni + p01r * udr - p01i * udi
        nLi = p00r * uni + p00i * unr + p01r * udi + p01i * udr
        dLr = p10r * unr - p10i * uni + p11r * udr - p11i * udi
        dLi = p10r * uni + p10i * unr + p11r * udi + p11i * udr
        dd = jnp.maximum(dLr * dLr + dLi * dLi, 1e-30)
        Lr = (nLr * dLr + nLi * dLi) / dd
        Li = (nLi * dLr - nLr * dLi) / dd
        (q00r, q00i, q01r, q01i, q10r, q10i, q11r, q11i) = Q[k]
        nRr = q00r * wnr - q00i * wni + q01r * wdr - q01i * wdi
        nRi = q00r * wni + q00i * wnr + q01r * wdi + q01i * wdr
        dRr = q10r * wnr - q10i * wni + q11r * wdr - q11i * wdi
        dRi = q10r * wni + q10i * wnr + q11r * wdi + q11i * wdr
        ddr = jnp.maximum(dRr * dRr + dRi * dRi, 1e-30)
        Rr = (nRr * dRr + nRi * dRi) / ddr
        Ri = (nRi * dRr - nRr * dRi) / ddr
        sr = Lr + Rr - ar[k:k + 1, :]
        si = Li + Ri + 1.0
        den = jnp.maximum(sr * sr + si * si, 1e-30)
        g_rows_r.append(jnp.clip(sr / den, -10.0, 10.0))
        g_rows_i.append(jnp.clip(-si / den, -10.0, 10.0))
    re_g = jnp.concatenate(g_rows_r, axis=0)
    im_g = jnp.concatenate(g_rows_i, axis=0)

    w1t = jnp.where((lax.broadcasted_iota(jnp.int32, (N, K), 0) % K)
                    == lax.broadcasted_iota(jnp.int32, (N, K), 1),
                    1.0, 0.0).astype(F32)
    col_r = jnp.sum(jnp.dot(w1t, re_g, preferred_element_type=F32) * sel_c,
                    axis=1, keepdims=True)
    col_i = jnp.sum(jnp.dot(w1t, im_g, preferred_element_type=F32) * sel_c,
                    axis=1, keepdims=True)
    spec = col_r * ow_ref[0:1, :] + col_i * ow_ref[1:2, :] + ob_ref[...]
    out_ref[...] = (moe + sc_ref[0, 0] * spec)[None]


def _bk_final(yu, gate, pw, pb, ow, ob, bscale):
    return pl.pallas_call(
        _bk_body,
        grid=(1,),
        in_specs=[
            pl.BlockSpec((N, D), lambda i: (0, 0)),
            pl.BlockSpec((N, 1), lambda i: (0, 0)),
            pl.BlockSpec((D, 1), lambda i: (0, 0)),
            pl.BlockSpec((1, 1), lambda i: (0, 0)),
            pl.BlockSpec((2, D), lambda i: (0, 0)),
            pl.BlockSpec((1, D), lambda i: (0, 0)),
            pl.BlockSpec((1, 1), lambda i: (0, 0)),
        ],
        out_specs=pl.BlockSpec((1, N, D), lambda i: (0, 0, 0)),
        out_shape=jax.ShapeDtypeStruct((1, N, D), F32),
    )(yu, gate, pw, pb, ow, ob, bscale)


def kernel(x, ln_gamma, ln_beta, router_w, router_b, w1, b1, w2, b2,
           pproj_w, pproj_b, oproj_w, oproj_b, bk_scale):
    x2 = x.reshape(N, D)
    g2, b2r = ln_gamma.reshape(1, D), ln_beta.reshape(1, D)
    gate, s, r_ids, e_ids, off = _ln_router(x2, g2, b2r, router_w,
                                            router_b.reshape(1, E))
    sf = s.reshape(N)
    xs = _sc_scatter(x2, sf)
    ys = _ffn_grouped(r_ids, e_ids, off, xs, g2, b2r, w1, b1, w2, b2)
    yu = _sc_gather(ys, sf)
    return _bk_final(yu, gate, pproj_w, pproj_b.reshape(1, 1), oproj_w,
                     oproj_b.reshape(1, D), jnp.asarray(bk_scale).reshape(1, 1))
